# baseline (speedup 1.0000x reference)
import jax
import jax.numpy as jnp
from jax.experimental import pallas as pl

B = 1024
FDIM = 128
SAMPLE_NUM = 8192
N = 100000
N_PAD = 100352
TILE = 1024


def _matmul_body(fn_ref, wn_ref, out_ref):
    out_ref[...] = jax.lax.dot_general(
        fn_ref[...], wn_ref[...],
        (((1,), (1,)), ((), ())),
        preferred_element_type=jnp.float32,
    )


def _scores(fn, wn_pad):
    grid = N_PAD // TILE
    return pl.pallas_call(
        _matmul_body,
        grid=(grid,),
        in_specs=[
            pl.BlockSpec((B, FDIM), lambda t: (0, 0)),
            pl.BlockSpec((TILE, FDIM), lambda t: (t, 0)),
        ],
        out_specs=pl.BlockSpec((B, TILE), lambda t: (0, t)),
        out_shape=jax.ShapeDtypeStruct((B, N_PAD), jnp.float32),
    )(fn, wn_pad)


def kernel(features, labels, weight):
    bsz = features.shape[0]
    n_nbr = SAMPLE_NUM // bsz + 1

    fn = features / (jnp.linalg.norm(features, axis=1, keepdims=True) + 1e-12)
    wn = weight / (jnp.linalg.norm(weight, axis=1, keepdims=True) + 1e-12)
    wn_pad = jnp.pad(wn, ((0, N_PAD - N), (0, 0)))

    scores = _scores(fn, wn_pad)[:, :N]

    _, topi = jax.lax.top_k(scores, n_nbr)
    cand = jnp.zeros((N,), jnp.float32).at[topi.reshape(-1)].set(1.0)
    pos = jnp.zeros((N,), jnp.float32).at[labels].set(1.0)

    uniq, inv = jnp.unique(labels, size=bsz, fill_value=0, return_inverse=True)
    inv = inv.reshape((bsz,)).astype(jnp.int64)

    agg = jnp.max(scores, axis=0)
    fallback = -1e30 - jnp.arange(N, dtype=jnp.float32)
    neg_score = jnp.where((cand > 0) & (pos == 0), agg,
                          jnp.where(pos > 0, jnp.full_like(agg, -1e35), fallback))
    rnum = SAMPLE_NUM - bsz
    neg_rows = jax.lax.top_k(neg_score, rnum)[1]

    rows = jnp.concatenate([uniq.astype(jnp.int32), neg_rows.astype(jnp.int32)])

    w_sel = jnp.take(weight, rows, axis=0)
    bias = jnp.zeros((SAMPLE_NUM,), jnp.float32)
    return w_sel, bias, inv


# speedup vs baseline: 3.3818x; 3.3818x over previous
import functools

import jax
import jax.numpy as jnp
from jax import lax
from jax.experimental import pallas as pl
from jax.experimental.pallas import tpu as pltpu
from jax.experimental.pallas import tpu_sc as plsc

B = 1024
FDIM = 128
SAMPLE_NUM = 8192
N = 100000
N_PAD = 100352
TILE = 1024
NBLK = N_PAD // 128
KSEL = 16
NNBR = SAMPLE_NUM // B + 1
NEG_INF = -3.0e38


def _scores_body(fn_ref, wn_ref, s_ref, m1t_ref, agg_ref):
    t = pl.program_id(0)
    s = jax.lax.dot_general(
        fn_ref[...], wn_ref[...],
        (((1,), (1,)), ((), ())),
        preferred_element_type=jnp.float32,
    )
    col = t * TILE + lax.broadcasted_iota(jnp.int32, (1, TILE), 1)
    s = jnp.where(col < N, s, NEG_INF)
    s_ref[...] = s
    maxes = [jnp.max(s[:, 128 * b:128 * (b + 1)], axis=1) for b in range(8)]
    m1t_ref[...] = jnp.stack(maxes, axis=0)
    agg_ref[...] = jnp.max(s, axis=0)


def _scores_call(fn, wn_pad):
    grid = N_PAD // TILE
    return pl.pallas_call(
        _scores_body,
        grid=(grid,),
        in_specs=[
            pl.BlockSpec((B, FDIM), lambda t: (0, 0)),
            pl.BlockSpec((TILE, FDIM), lambda t: (t, 0)),
        ],
        out_specs=[
            pl.BlockSpec((B, TILE), lambda t: (0, t)),
            pl.BlockSpec((8, B), lambda t: (t, 0)),
            pl.BlockSpec((TILE,), lambda t: (t,)),
        ],
        out_shape=[
            jax.ShapeDtypeStruct((B, N_PAD), jnp.float32),
            jax.ShapeDtypeStruct((NBLK, B), jnp.float32),
            jax.ShapeDtypeStruct((N_PAD,), jnp.float32),
        ],
    )(fn, wn_pad)


def _blocksel_body(m1t_ref, sel_ref):
    v = m1t_ref[...]
    blk = lax.broadcasted_iota(jnp.int32, (NBLK, B), 0)
    rows = []
    for _ in range(KSEL):
        m = jnp.max(v, axis=0, keepdims=True)
        bi = jnp.min(jnp.where(v == m, blk, NBLK), axis=0, keepdims=True)
        rows.append(bi)
        v = jnp.where(blk == bi, NEG_INF, v)
    q = lax.broadcasted_iota(jnp.int32, (KSEL, B), 1)
    sel_ref[...] = jnp.concatenate(rows, axis=0) + q * NBLK


def _blocksel_call(m1t):
    return pl.pallas_call(
        _blocksel_body,
        in_specs=[pl.BlockSpec((NBLK, B), lambda: (0, 0))],
        out_specs=pl.BlockSpec((KSEL, B), lambda: (0, 0)),
        out_shape=jax.ShapeDtypeStruct((KSEL, B), jnp.int32),
    )(m1t)


def _sc_gather(table, idx, rows_out_shape):
    info = plsc.get_sparse_core_info()
    nw = info.num_cores * info.num_subcores
    n_idx = idx.shape[0]
    d = table.shape[1]
    b_per_w = n_idx // nw
    mesh = plsc.VectorSubcoreMesh(core_axis_name="c", subcore_axis_name="s")

    @functools.partial(
        pl.kernel, mesh=mesh,
        out_type=jax.ShapeDtypeStruct(rows_out_shape, table.dtype),
        scratch_types=[
            pltpu.VMEM((b_per_w,), jnp.int32),
            pltpu.VMEM((b_per_w, d), table.dtype),
            pltpu.SemaphoreType.DMA,
        ],
    )
    def k(table_hbm, idx_hbm, out_hbm, idx_v, rows_v, sem):
        wid = lax.axis_index("s") * info.num_cores + lax.axis_index("c")
        base = wid * b_per_w
        pltpu.sync_copy(idx_hbm.at[pl.ds(base, b_per_w)], idx_v)
        pltpu.async_copy(table_hbm.at[idx_v], rows_v, sem).wait()
        pltpu.sync_copy(rows_v, out_hbm.at[pl.ds(base, b_per_w)])

    return k(table, idx)


def _top9_body(g_ref, bsel_ref, out_ref):
    v = g_ref[...]
    lane = lax.broadcasted_iota(jnp.int32, (1, 128), 1)
    chunks = []
    for k in range(KSEL):
        b = bsel_ref[:, k:k + 1] % NBLK
        chunks.append(b * 128 + lane)
    gidx = jnp.concatenate(chunks, axis=1)
    outs = []
    for r in range(NNBR):
        m = jnp.max(v, axis=1, keepdims=True)
        g = jnp.min(jnp.where(v == m, gidx, N_PAD), axis=1, keepdims=True)
        outs.append(g)
        v = jnp.where(gidx == g, NEG_INF, v)
    outs += [outs[0]] * (KSEL - NNBR)
    out_ref[...] = jnp.concatenate(outs, axis=1)


def _top9_call(gath2d, bsel):
    return pl.pallas_call(
        _top9_body,
        in_specs=[
            pl.BlockSpec((B, KSEL * 128), lambda: (0, 0)),
            pl.BlockSpec((B, KSEL), lambda: (0, 0)),
        ],
        out_specs=pl.BlockSpec((B, KSEL), lambda: (0, 0)),
        out_shape=jax.ShapeDtypeStruct((B, KSEL), jnp.int32),
    )(gath2d, bsel)


RNUM = SAMPLE_NUM - B
NCAND = KSEL * B
CPAD = 9216


def _prefix_body(cand_ref, pos_ref, lab2_ref, lab_ref, p_ref, q_ref,
                 inv_ref, fo_ref):
    cand = cand_ref[...]
    pos = pos_ref[...]
    candnp = cand * (1.0 - pos)
    jr = lax.broadcasted_iota(jnp.int32, (NBLK, 128), 0)
    jc = lax.broadcasted_iota(jnp.int32, (NBLK, 128), 1)
    valid = (jr * 128 + jc) < N
    fb = jnp.where(valid, (1.0 - cand) * (1.0 - pos), 0.0)

    ia = lax.broadcasted_iota(jnp.int32, (NBLK, NBLK), 0)
    ib = lax.broadcasted_iota(jnp.int32, (NBLK, NBLK), 1)
    slt = (ia > ib).astype(jnp.float32)
    ua = lax.broadcasted_iota(jnp.int32, (128, 128), 0)
    ub = lax.broadcasted_iota(jnp.int32, (128, 128), 1)
    su = (ua < ub).astype(jnp.float32)
    ones = jnp.ones((128, 128), jnp.float32)

    def mm(a, b):
        return jax.lax.dot_general(a, b, (((1,), (0,)), ((), ())),
                                   preferred_element_type=jnp.float32)

    p_ref[...] = mm(slt, mm(candnp, ones)) + mm(candnp, su)
    q_ref[...] = mm(slt, mm(fb, ones)) + mm(fb, su)

    la = lab2_ref[...]
    lb = lab_ref[...].reshape(1, B)
    pa = lax.broadcasted_iota(jnp.int32, (B, B), 0)
    pb = lax.broadcasted_iota(jnp.int32, (B, B), 1)
    earlier = ((la == lb) & (pa < pb)).astype(jnp.float32)
    fo = jnp.sum(earlier, axis=0) == 0.0
    lt = (la < lb).astype(jnp.float32)
    fo8 = jnp.broadcast_to(fo.astype(jnp.float32).reshape(1, B), (8, B))
    invm = mm(fo8, lt)
    inv_ref[...] = invm[0].astype(jnp.int32)
    fo_ref[...] = fo.astype(jnp.int32)


def _prefix_call(cand2d, pos2d, labels2d, labels):
    return pl.pallas_call(
        _prefix_body,
        in_specs=[
            pl.BlockSpec((NBLK, 128), lambda: (0, 0)),
            pl.BlockSpec((NBLK, 128), lambda: (0, 0)),
            pl.BlockSpec((B, 1), lambda: (0, 0)),
            pl.BlockSpec((B,), lambda: (0,)),
        ],
        out_specs=[
            pl.BlockSpec((NBLK, 128), lambda: (0, 0)),
            pl.BlockSpec((NBLK, 128), lambda: (0, 0)),
            pl.BlockSpec((B,), lambda: (0,)),
            pl.BlockSpec((B,), lambda: (0,)),
        ],
        out_shape=[
            jax.ShapeDtypeStruct((NBLK, 128), jnp.float32),
            jax.ShapeDtypeStruct((NBLK, 128), jnp.float32),
            jax.ShapeDtypeStruct((B,), jnp.int32),
            jax.ShapeDtypeStruct((B,), jnp.int32),
        ],
    )(cand2d, pos2d, labels2d, labels)


GI = 512


def _rank_body(a2_ref, i2_ref, af_ref, if_ref, r_ref):
    ai = a2_ref[...]
    ii = i2_ref[...]
    aj = af_ref[...].reshape(1, CPAD)
    ij = if_ref[...].reshape(1, CPAD)
    gt = (aj > ai) | ((aj == ai) & (ij < ii))
    r_ref[...] = jnp.sum(gt.astype(jnp.float32), axis=1,
                         keepdims=True).astype(jnp.int32)


def _rank_call(aggc, candidx):
    return pl.pallas_call(
        _rank_body,
        grid=(CPAD // GI,),
        in_specs=[
            pl.BlockSpec((GI, 1), lambda t: (t, 0)),
            pl.BlockSpec((GI, 1), lambda t: (t, 0)),
            pl.BlockSpec((CPAD,), lambda t: (0,)),
            pl.BlockSpec((CPAD,), lambda t: (0,)),
        ],
        out_specs=pl.BlockSpec((GI, 1), lambda t: (t, 0)),
        out_shape=jax.ShapeDtypeStruct((CPAD, 1), jnp.int32),
    )(aggc.reshape(CPAD, 1), candidx.reshape(CPAD, 1), aggc, candidx)


def _sc_masks(topidx_flat, labels):
    mesh = plsc.VectorSubcoreMesh(core_axis_name="c", subcore_axis_name="s")
    info = plsc.get_sparse_core_info()

    @functools.partial(
        pl.kernel, mesh=mesh,
        compiler_params=pltpu.CompilerParams(needs_layout_passes=False),
        out_type=[
            jax.ShapeDtypeStruct((N_PAD,), jnp.float32),
            jax.ShapeDtypeStruct((N_PAD,), jnp.float32),
        ],
        scratch_types=[
            pltpu.VMEM((N_PAD,), jnp.float32),
            pltpu.VMEM((NCAND,), jnp.int32),
        ],
    )
    def k(ti_hbm, lab_hbm, cand_hbm, pos_hbm, mask_v, idx_v):
        wid = lax.axis_index("s") * info.num_cores + lax.axis_index("c")
        ones16 = jnp.ones((16,), jnp.float32)

        @pl.when(wid == 0)
        def _():
            def zf(i, _):
                mask_v[pl.ds(i * 16, 16)] = jnp.zeros((16,), jnp.float32)
                return 0
            lax.fori_loop(0, N_PAD // 16, zf, 0)
            pltpu.sync_copy(ti_hbm, idx_v)

            def sc(i, _):
                iv = idx_v[pl.ds(i * 16, 16)]
                plsc.store_scatter(mask_v, [iv], ones16)
                return 0
            lax.fori_loop(0, NCAND // 16, sc, 0)
            pltpu.sync_copy(mask_v, cand_hbm)

        @pl.when(wid == 1)
        def _():
            def zf(i, _):
                mask_v[pl.ds(i * 16, 16)] = jnp.zeros((16,), jnp.float32)
                return 0
            lax.fori_loop(0, N_PAD // 16, zf, 0)
            pltpu.sync_copy(lab_hbm, idx_v.at[pl.ds(0, B)])

            def sc(i, _):
                iv = idx_v[pl.ds(i * 16, 16)]
                plsc.store_scatter(mask_v, [iv], ones16)
                return 0
            lax.fori_loop(0, B // 16, sc, 0)
            pltpu.sync_copy(mask_v, pos_hbm)

    return k(topidx_flat, labels)


FCH = 7168
NCHUNK = N_PAD // FCH


def _sc_compact(cand, pos, pflat, qflat, agg, labels, fo, inv, cvec, fvec):
    mesh = plsc.VectorSubcoreMesh(core_axis_name="c", subcore_axis_name="s")
    info = plsc.get_sparse_core_info()

    @functools.partial(
        pl.kernel, mesh=mesh,
        compiler_params=pltpu.CompilerParams(needs_layout_passes=False),
        out_type=[
            jax.ShapeDtypeStruct((CPAD,), jnp.float32),
            jax.ShapeDtypeStruct((CPAD,), jnp.int32),
            jax.ShapeDtypeStruct((SAMPLE_NUM,), jnp.int32),
        ],
        scratch_types=[
            pltpu.VMEM((CPAD,), jnp.float32),
            pltpu.VMEM((CPAD,), jnp.int32),
            pltpu.VMEM((SAMPLE_NUM,), jnp.int32),
            pltpu.VMEM((FCH,), jnp.float32),
            pltpu.VMEM((FCH,), jnp.float32),
            pltpu.VMEM((FCH,), jnp.int32),
            pltpu.VMEM((FCH,), jnp.int32),
            pltpu.VMEM((FCH,), jnp.float32),
            pltpu.VMEM((B,), jnp.int32),
            pltpu.VMEM((B,), jnp.int32),
            pltpu.VMEM((B,), jnp.int32),
            pltpu.VMEM((16,), jnp.int32),
            pltpu.VMEM((16,), jnp.int32),
        ],
    )
    def k(cand_h, pos_h, p_h, q_h, agg_h, lab_h, fo_h, inv_h, cv_h, fv_h,
          aggc_o, candidx_o, rows_o,
          aggc_v, cidx_v, rows_v, cc_v, pc_v, pp_v, qq_v, ac_v,
          lab_v, fo_v, inv_v, cvec_v, fvec_v):
        wid = lax.axis_index("s") * info.num_cores + lax.axis_index("c")

        @pl.when(wid == 0)
        def _():
            def init(i, _):
                aggc_v[pl.ds(i * 16, 16)] = jnp.full((16,), NEG_INF, jnp.float32)
                cidx_v[pl.ds(i * 16, 16)] = jnp.full((16,), -1, jnp.int32)
                return 0
            lax.fori_loop(0, CPAD // 16, init, 0)

            def zrow(i, _):
                rows_v[pl.ds(i * 16, 16)] = jnp.zeros((16,), jnp.int32)
                return 0
            lax.fori_loop(0, SAMPLE_NUM // 16, zrow, 0)

            pltpu.sync_copy(lab_h, lab_v)
            pltpu.sync_copy(fo_h, fo_v)
            pltpu.sync_copy(inv_h, inv_v)
            pltpu.sync_copy(cv_h, cvec_v)
            pltpu.sync_copy(fv_h, fvec_v)

            def usc(i, _):
                lab = lab_v[pl.ds(i * 16, 16)]
                dst = inv_v[pl.ds(i * 16, 16)]
                m = fo_v[pl.ds(i * 16, 16)] != 0
                plsc.store_scatter(rows_v, [jnp.minimum(dst, B - 1)], lab,
                                   mask=m)
                return 0
            lax.fori_loop(0, B // 16, usc, 0)

            cvec = cvec_v[pl.ds(0, 16)]
            fvec = fvec_v[pl.ds(0, 16)]
            for c in range(NCHUNK):
                pltpu.sync_copy(cand_h.at[pl.ds(c * FCH, FCH)], cc_v)
                pltpu.sync_copy(pos_h.at[pl.ds(c * FCH, FCH)], pc_v)
                pltpu.sync_copy(p_h.at[pl.ds(c * FCH, FCH)], pp_v)
                pltpu.sync_copy(q_h.at[pl.ds(c * FCH, FCH)], qq_v)
                pltpu.sync_copy(agg_h.at[pl.ds(c * FCH, FCH)], ac_v)
                base = c * FCH

                def sweep(i, _):
                    jv = lax.iota(jnp.int32, 16) + (base + i * 16)
                    cv = cc_v[pl.ds(i * 16, 16)]
                    pv = pc_v[pl.ds(i * 16, 16)]
                    Pv = pp_v[pl.ds(i * 16, 16)]
                    Qv = qq_v[pl.ds(i * 16, 16)]
                    av = ac_v[pl.ds(i * 16, 16)]
                    isc = (cv > 0.0) & (pv == 0.0)
                    Pc = jnp.minimum(Pv, CPAD - 1)
                    plsc.store_scatter(aggc_v, [Pc], av, mask=isc)
                    plsc.store_scatter(cidx_v, [Pc], jv, mask=isc)
                    fbm = ((cv == 0.0) & (pv == 0.0) & (jv < N)
                           & (Qv < fvec))
                    dst = jnp.minimum(cvec + Qv, SAMPLE_NUM - 1)
                    plsc.store_scatter(rows_v, [dst], jv, mask=fbm)
                    return 0
                lax.fori_loop(0, FCH // 16, sweep, 0)

            pltpu.sync_copy(aggc_v, aggc_o)
            pltpu.sync_copy(cidx_v, candidx_o)
            pltpu.sync_copy(rows_v, rows_o)

    return k(cand, pos, pflat, qflat, agg, labels, fo, inv, cvec, fvec)


def _sc_rows(rows_part, rank, candidx):
    mesh = plsc.VectorSubcoreMesh(core_axis_name="c", subcore_axis_name="s")
    info = plsc.get_sparse_core_info()

    @functools.partial(
        pl.kernel, mesh=mesh,
        compiler_params=pltpu.CompilerParams(needs_layout_passes=False),
        out_type=jax.ShapeDtypeStruct((SAMPLE_NUM,), jnp.int32),
        scratch_types=[
            pltpu.VMEM((SAMPLE_NUM,), jnp.int32),
            pltpu.VMEM((CPAD,), jnp.int32),
            pltpu.VMEM((CPAD,), jnp.int32),
        ],
    )
    def k(rp_h, rk_h, ci_h, rows_o, rows_v, rk_v, ci_v):
        wid = lax.axis_index("s") * info.num_cores + lax.axis_index("c")

        @pl.when(wid == 0)
        def _():
            pltpu.sync_copy(rp_h, rows_v)
            pltpu.sync_copy(rk_h, rk_v)
            pltpu.sync_copy(ci_h, ci_v)

            def sc(i, _):
                rv = rk_v[pl.ds(i * 16, 16)]
                cv = ci_v[pl.ds(i * 16, 16)]
                m = (rv < RNUM) & (cv >= 0)
                dst = B + jnp.minimum(rv, RNUM - 1)
                plsc.store_scatter(rows_v, [dst], cv, mask=m)
                return 0
            lax.fori_loop(0, CPAD // 16, sc, 0)
            pltpu.sync_copy(rows_v, rows_o)

    return k(rows_part, rank, candidx)


def kernel(features, labels, weight):
    bsz = features.shape[0]

    fn = features / (jnp.linalg.norm(features, axis=1, keepdims=True) + 1e-12)
    wn = weight / (jnp.linalg.norm(weight, axis=1, keepdims=True) + 1e-12)
    wn_pad = jnp.pad(wn, ((0, N_PAD - N), (0, 0)))

    scores, m1t, agg = _scores_call(fn, wn_pad)

    selT = _blocksel_call(m1t)
    sel = selT.T
    gath = _sc_gather(scores.reshape(B * NBLK, 128), sel.reshape(-1),
                      (B * KSEL, 128))
    topidx = _top9_call(gath.reshape(B, KSEL * 128), sel)

    cand, pos = _sc_masks(topidx.reshape(-1), labels)

    p2d, q2d, inv, fo = _prefix_call(cand.reshape(NBLK, 128),
                                     pos.reshape(NBLK, 128),
                                     labels.reshape(B, 1), labels)
    pflat = p2d.reshape(-1).astype(jnp.int32)
    qflat = q2d.reshape(-1).astype(jnp.int32)
    c_tot = pflat[-1]
    f_tot = jnp.maximum(RNUM - c_tot, 0)
    cvec = jnp.full((16,), B, jnp.int32) + c_tot
    fvec = jnp.full((16,), 0, jnp.int32) + f_tot

    aggc, candidx, rows_part = _sc_compact(
        cand, pos, pflat, qflat, agg, labels, fo, inv, cvec, fvec)

    rank = _rank_call(aggc, candidx).reshape(-1)

    rows = _sc_rows(rows_part, rank, candidx)

    w_sel = _sc_gather(weight, rows, (SAMPLE_NUM, FDIM))
    bias = jnp.zeros((SAMPLE_NUM,), jnp.float32)
    return w_sel, bias, inv.astype(jnp.int64)


# speedup vs baseline: 5.2598x; 1.5553x over previous
import functools

import jax
import jax.numpy as jnp
from jax import lax
from jax.experimental import pallas as pl
from jax.experimental.pallas import tpu as pltpu
from jax.experimental.pallas import tpu_sc as plsc

B = 1024
FDIM = 128
SAMPLE_NUM = 8192
N = 100000
N_PAD = 100352
TILE = 1024
NBLK = N_PAD // 128
KSEL = 16
NNBR = SAMPLE_NUM // B + 1
NEG_INF = -3.0e38


def _scores_body(fn_ref, wn_ref, s_ref, m1t_ref, agg_ref):
    t = pl.program_id(0)
    s = jax.lax.dot_general(
        fn_ref[...], wn_ref[...],
        (((1,), (1,)), ((), ())),
        preferred_element_type=jnp.float32,
    )
    col = t * TILE + lax.broadcasted_iota(jnp.int32, (1, TILE), 1)
    s = jnp.where(col < N, s, NEG_INF)
    s_ref[...] = jnp.stack([s[:, 128 * b:128 * (b + 1)] for b in range(8)],
                           axis=0)
    maxes = [jnp.max(s[:, 128 * b:128 * (b + 1)], axis=1) for b in range(8)]
    m1t_ref[...] = jnp.stack(maxes, axis=0)
    agg_ref[...] = jnp.max(s, axis=0)


def _scores_call(fn, wn_pad):
    grid = N_PAD // TILE
    return pl.pallas_call(
        _scores_body,
        grid=(grid,),
        in_specs=[
            pl.BlockSpec((B, FDIM), lambda t: (0, 0)),
            pl.BlockSpec((TILE, FDIM), lambda t: (t, 0)),
        ],
        out_specs=[
            pl.BlockSpec((8, B, 128), lambda t: (t, 0, 0)),
            pl.BlockSpec((8, B), lambda t: (t, 0)),
            pl.BlockSpec((TILE,), lambda t: (t,)),
        ],
        out_shape=[
            jax.ShapeDtypeStruct((NBLK, B, 128), jnp.float32),
            jax.ShapeDtypeStruct((NBLK, B), jnp.float32),
            jax.ShapeDtypeStruct((N_PAD,), jnp.float32),
        ],
    )(fn, wn_pad)


def _blocksel_body(m1t_ref, sel_ref):
    v = m1t_ref[...]
    blk = lax.broadcasted_iota(jnp.int32, (NBLK, B), 0)
    rows = []
    for _ in range(KSEL):
        m = jnp.max(v, axis=0, keepdims=True)
        bi = jnp.min(jnp.where(v == m, blk, NBLK), axis=0, keepdims=True)
        rows.append(bi)
        v = jnp.where(blk == bi, NEG_INF, v)
    q = lax.broadcasted_iota(jnp.int32, (KSEL, B), 1)
    sel_ref[...] = jnp.concatenate(rows, axis=0) * B + q


def _blocksel_call(m1t):
    return pl.pallas_call(
        _blocksel_body,
        in_specs=[pl.BlockSpec((NBLK, B), lambda: (0, 0))],
        out_specs=pl.BlockSpec((KSEL, B), lambda: (0, 0)),
        out_shape=jax.ShapeDtypeStruct((KSEL, B), jnp.int32),
    )(m1t)


def _sc_gather(table, idx, rows_out_shape):
    info = plsc.get_sparse_core_info()
    nw = info.num_cores * info.num_subcores
    n_idx = idx.shape[0]
    d = table.shape[1]
    b_per_w = n_idx // nw
    mesh = plsc.VectorSubcoreMesh(core_axis_name="c", subcore_axis_name="s")

    @functools.partial(
        pl.kernel, mesh=mesh,
        out_type=jax.ShapeDtypeStruct(rows_out_shape, table.dtype),
        scratch_types=[
            pltpu.VMEM((b_per_w,), jnp.int32),
            pltpu.VMEM((b_per_w, d), table.dtype),
            pltpu.SemaphoreType.DMA,
        ],
    )
    def k(table_hbm, idx_hbm, out_hbm, idx_v, rows_v, sem):
        wid = lax.axis_index("s") * info.num_cores + lax.axis_index("c")
        base = wid * b_per_w
        pltpu.sync_copy(idx_hbm.at[pl.ds(base, b_per_w)], idx_v)
        pltpu.async_copy(table_hbm.at[idx_v], rows_v, sem).wait()
        pltpu.sync_copy(rows_v, out_hbm.at[pl.ds(base, b_per_w)])

    return k(table, idx)


def _top9_body(g_ref, bsel_ref, out_ref):
    v = g_ref[...]
    lane = lax.broadcasted_iota(jnp.int32, (1, 128), 1)
    chunks = []
    for k in range(KSEL):
        b = bsel_ref[:, k:k + 1] // B
        chunks.append(b * 128 + lane)
    gidx = jnp.concatenate(chunks, axis=1)
    outs = []
    for r in range(NNBR):
        m = jnp.max(v, axis=1, keepdims=True)
        g = jnp.min(jnp.where(v == m, gidx, N_PAD), axis=1, keepdims=True)
        outs.append(g)
        v = jnp.where(gidx == g, NEG_INF, v)
    outs += [outs[0]] * (KSEL - NNBR)
    out_ref[...] = jnp.concatenate(outs, axis=1)


def _top9_call(gath2d, bsel):
    return pl.pallas_call(
        _top9_body,
        in_specs=[
            pl.BlockSpec((B, KSEL * 128), lambda: (0, 0)),
            pl.BlockSpec((B, KSEL), lambda: (0, 0)),
        ],
        out_specs=pl.BlockSpec((B, KSEL), lambda: (0, 0)),
        out_shape=jax.ShapeDtypeStruct((B, KSEL), jnp.int32),
    )(gath2d, bsel)


RNUM = SAMPLE_NUM - B
NCAND = KSEL * B
CPAD = 9216


def _prefix_body(cand_ref, pos_ref, lab2_ref, lab_ref, p_ref, q_ref,
                 inv_ref, fo_ref):
    cand = cand_ref[...]
    pos = pos_ref[...]
    candnp = cand * (1.0 - pos)
    jr = lax.broadcasted_iota(jnp.int32, (NBLK, 128), 0)
    jc = lax.broadcasted_iota(jnp.int32, (NBLK, 128), 1)
    valid = (jr * 128 + jc) < N
    fb = jnp.where(valid, (1.0 - cand) * (1.0 - pos), 0.0)

    ia = lax.broadcasted_iota(jnp.int32, (NBLK, NBLK), 0)
    ib = lax.broadcasted_iota(jnp.int32, (NBLK, NBLK), 1)
    slt = (ia > ib).astype(jnp.float32)
    ua = lax.broadcasted_iota(jnp.int32, (128, 128), 0)
    ub = lax.broadcasted_iota(jnp.int32, (128, 128), 1)
    su = (ua < ub).astype(jnp.float32)
    ones = jnp.ones((128, 128), jnp.float32)

    def mm(a, b):
        return jax.lax.dot_general(a, b, (((1,), (0,)), ((), ())),
                                   preferred_element_type=jnp.float32)

    p_ref[...] = mm(slt, mm(candnp, ones)) + mm(candnp, su)
    q_ref[...] = mm(slt, mm(fb, ones)) + mm(fb, su)

    la = lab2_ref[...]
    lb = lab_ref[...].reshape(1, B)
    pa = lax.broadcasted_iota(jnp.int32, (B, B), 0)
    pb = lax.broadcasted_iota(jnp.int32, (B, B), 1)
    earlier = ((la == lb) & (pa < pb)).astype(jnp.float32)
    fo = jnp.sum(earlier, axis=0) == 0.0
    lt = (la < lb).astype(jnp.float32)
    fo8 = jnp.broadcast_to(fo.astype(jnp.float32).reshape(1, B), (8, B))
    invm = mm(fo8, lt)
    inv_ref[...] = invm[0].astype(jnp.int32)
    fo_ref[...] = fo.astype(jnp.int32)


def _prefix_call(cand2d, pos2d, labels2d, labels):
    return pl.pallas_call(
        _prefix_body,
        in_specs=[
            pl.BlockSpec((NBLK, 128), lambda: (0, 0)),
            pl.BlockSpec((NBLK, 128), lambda: (0, 0)),
            pl.BlockSpec((B, 1), lambda: (0, 0)),
            pl.BlockSpec((B,), lambda: (0,)),
        ],
        out_specs=[
            pl.BlockSpec((NBLK, 128), lambda: (0, 0)),
            pl.BlockSpec((NBLK, 128), lambda: (0, 0)),
            pl.BlockSpec((B,), lambda: (0,)),
            pl.BlockSpec((B,), lambda: (0,)),
        ],
        out_shape=[
            jax.ShapeDtypeStruct((NBLK, 128), jnp.float32),
            jax.ShapeDtypeStruct((NBLK, 128), jnp.float32),
            jax.ShapeDtypeStruct((B,), jnp.int32),
            jax.ShapeDtypeStruct((B,), jnp.int32),
        ],
    )(cand2d, pos2d, labels2d, labels)


GI = 512


def _rank_body(a2_ref, i2_ref, af_ref, if_ref, r_ref):
    ai = a2_ref[...]
    ii = i2_ref[...]
    aj = af_ref[...].reshape(1, CPAD)
    ij = if_ref[...].reshape(1, CPAD)
    gt = (aj > ai) | ((aj == ai) & (ij < ii))
    r_ref[...] = jnp.sum(gt.astype(jnp.float32), axis=1,
                         keepdims=True).astype(jnp.int32)


def _rank_call(aggc, candidx):
    return pl.pallas_call(
        _rank_body,
        grid=(CPAD // GI,),
        in_specs=[
            pl.BlockSpec((GI, 1), lambda t: (t, 0)),
            pl.BlockSpec((GI, 1), lambda t: (t, 0)),
            pl.BlockSpec((CPAD,), lambda t: (0,)),
            pl.BlockSpec((CPAD,), lambda t: (0,)),
        ],
        out_specs=pl.BlockSpec((GI, 1), lambda t: (t, 0)),
        out_shape=jax.ShapeDtypeStruct((CPAD, 1), jnp.int32),
    )(aggc.reshape(CPAD, 1), candidx.reshape(CPAD, 1), aggc, candidx)


def _sc_masks(topidx_flat, labels):
    mesh = plsc.VectorSubcoreMesh(core_axis_name="c", subcore_axis_name="s")
    info = plsc.get_sparse_core_info()

    @functools.partial(
        pl.kernel, mesh=mesh,
        compiler_params=pltpu.CompilerParams(needs_layout_passes=False),
        out_type=[
            jax.ShapeDtypeStruct((N_PAD,), jnp.float32),
            jax.ShapeDtypeStruct((N_PAD,), jnp.float32),
        ],
        scratch_types=[
            pltpu.VMEM((N_PAD,), jnp.float32),
            pltpu.VMEM((NCAND,), jnp.int32),
        ],
    )
    def k(ti_hbm, lab_hbm, cand_hbm, pos_hbm, mask_v, idx_v):
        wid = lax.axis_index("s") * info.num_cores + lax.axis_index("c")
        ones16 = jnp.ones((16,), jnp.float32)

        @pl.when(wid == 0)
        def _():
            def zf(i, _):
                mask_v[pl.ds(i * 16, 16)] = jnp.zeros((16,), jnp.float32)
                return 0
            lax.fori_loop(0, N_PAD // 16, zf, 0)
            pltpu.sync_copy(ti_hbm, idx_v)

            def sc(i, _):
                iv = idx_v[pl.ds(i * 16, 16)]
                plsc.store_scatter(mask_v, [iv], ones16)
                return 0
            lax.fori_loop(0, NCAND // 16, sc, 0)
            pltpu.sync_copy(mask_v, cand_hbm)

        @pl.when(wid == 1)
        def _():
            def zf(i, _):
                mask_v[pl.ds(i * 16, 16)] = jnp.zeros((16,), jnp.float32)
                return 0
            lax.fori_loop(0, N_PAD // 16, zf, 0)
            pltpu.sync_copy(lab_hbm, idx_v.at[pl.ds(0, B)])

            def sc(i, _):
                iv = idx_v[pl.ds(i * 16, 16)]
                plsc.store_scatter(mask_v, [iv], ones16)
                return 0
            lax.fori_loop(0, B // 16, sc, 0)
            pltpu.sync_copy(mask_v, pos_hbm)

    return k(topidx_flat, labels)


FCH = 14336
NCHUNK = N_PAD // FCH


def _sc_compact(cand, pos, pflat, qflat, agg, labels, fo, inv, cvec, fvec):
    mesh = plsc.VectorSubcoreMesh(core_axis_name="c", subcore_axis_name="s")
    info = plsc.get_sparse_core_info()

    @functools.partial(
        pl.kernel, mesh=mesh,
        compiler_params=pltpu.CompilerParams(needs_layout_passes=False),
        out_type=[
            jax.ShapeDtypeStruct((CPAD,), jnp.float32),
            jax.ShapeDtypeStruct((CPAD,), jnp.int32),
            jax.ShapeDtypeStruct((SAMPLE_NUM,), jnp.int32),
        ],
        scratch_types=[
            pltpu.VMEM((CPAD,), jnp.float32),
            pltpu.VMEM((CPAD,), jnp.int32),
            pltpu.VMEM((SAMPLE_NUM,), jnp.int32),
            pltpu.VMEM((FCH,), jnp.float32),
            pltpu.VMEM((FCH,), jnp.float32),
            pltpu.VMEM((FCH,), jnp.int32),
            pltpu.VMEM((FCH,), jnp.float32),
            pltpu.VMEM((B,), jnp.int32),
            pltpu.VMEM((B,), jnp.int32),
            pltpu.VMEM((B,), jnp.int32),
            pltpu.VMEM((16,), jnp.int32),
            pltpu.VMEM((16,), jnp.int32),
            pltpu.SemaphoreType.DMA,
        ],
    )
    def k(cand_h, pos_h, p_h, q_h, agg_h, lab_h, fo_h, inv_h, cv_h, fv_h,
          aggc_o, candidx_o, rows_o,
          aggc_v, cidx_v, rows_v, cc_v, pc_v, pq_v, ac_v,
          lab_v, fo_v, inv_v, cvec_v, fvec_v, sem):
        wid = lax.axis_index("s") * info.num_cores + lax.axis_index("c")

        @pl.when(wid == 0)
        def _():
            def init(i, _):
                aggc_v[pl.ds(i * 16, 16)] = jnp.full((16,), NEG_INF, jnp.float32)
                cidx_v[pl.ds(i * 16, 16)] = jnp.full((16,), -1, jnp.int32)
                return 0
            lax.fori_loop(0, CPAD // 16, init, 0)

            for c in range(NCHUNK):
                sl = pl.ds(c * FCH, FCH)
                h1 = pltpu.async_copy(cand_h.at[sl], cc_v, sem)
                h2 = pltpu.async_copy(pos_h.at[sl], pc_v, sem)
                h3 = pltpu.async_copy(p_h.at[sl], pq_v, sem)
                h4 = pltpu.async_copy(agg_h.at[sl], ac_v, sem)
                h1.wait(); h2.wait(); h3.wait(); h4.wait()
                base = c * FCH

                def sweep(i, _):
                    jv = lax.iota(jnp.int32, 16) + (base + i * 16)
                    cv = cc_v[pl.ds(i * 16, 16)]
                    pv = pc_v[pl.ds(i * 16, 16)]
                    Pv = pq_v[pl.ds(i * 16, 16)]
                    av = ac_v[pl.ds(i * 16, 16)]
                    isc = (cv > 0.0) & (pv == 0.0)
                    Pc = jnp.minimum(Pv, CPAD - 1)
                    plsc.store_scatter(aggc_v, [Pc], av, mask=isc)
                    plsc.store_scatter(cidx_v, [Pc], jv, mask=isc)
                    return 0
                lax.fori_loop(0, FCH // 16, sweep, 0)

            pltpu.sync_copy(aggc_v, aggc_o)
            pltpu.sync_copy(cidx_v, candidx_o)

        @pl.when(wid == 1)
        def _():
            def zrow(i, _):
                rows_v[pl.ds(i * 16, 16)] = jnp.zeros((16,), jnp.int32)
                return 0
            lax.fori_loop(0, B // 16, zrow, 0)

            pltpu.sync_copy(lab_h, lab_v)
            pltpu.sync_copy(fo_h, fo_v)
            pltpu.sync_copy(inv_h, inv_v)
            pltpu.sync_copy(cv_h, cvec_v)
            pltpu.sync_copy(fv_h, fvec_v)

            def usc(i, _):
                lab = lab_v[pl.ds(i * 16, 16)]
                dst = inv_v[pl.ds(i * 16, 16)]
                m = fo_v[pl.ds(i * 16, 16)] != 0
                plsc.store_scatter(rows_v, [jnp.minimum(dst, B - 1)], lab,
                                   mask=m)
                return 0
            lax.fori_loop(0, B // 16, usc, 0)

            cvec = cvec_v[pl.ds(0, 16)]
            fvec = fvec_v[pl.ds(0, 16)]
            for c in range(NCHUNK):
                sl = pl.ds(c * FCH, FCH)
                h1 = pltpu.async_copy(cand_h.at[sl], cc_v, sem)
                h2 = pltpu.async_copy(pos_h.at[sl], pc_v, sem)
                h3 = pltpu.async_copy(q_h.at[sl], pq_v, sem)
                h1.wait(); h2.wait(); h3.wait()
                base = c * FCH

                def fsw(i, _):
                    jv = lax.iota(jnp.int32, 16) + (base + i * 16)
                    cv = cc_v[pl.ds(i * 16, 16)]
                    pv = pc_v[pl.ds(i * 16, 16)]
                    Qv = pq_v[pl.ds(i * 16, 16)]
                    fbm = ((cv == 0.0) & (pv == 0.0) & (jv < N)
                           & (Qv < fvec))
                    dst = jnp.minimum(cvec + Qv, SAMPLE_NUM - 1)
                    plsc.store_scatter(rows_v, [dst], jv, mask=fbm)
                    return 0
                lax.fori_loop(0, FCH // 16, fsw, 0)

            pltpu.sync_copy(rows_v, rows_o)

    return k(cand, pos, pflat, qflat, agg, labels, fo, inv, cvec, fvec)


def _sc_rows(rows_part, rank, candidx):
    mesh = plsc.VectorSubcoreMesh(core_axis_name="c", subcore_axis_name="s")
    info = plsc.get_sparse_core_info()

    @functools.partial(
        pl.kernel, mesh=mesh,
        compiler_params=pltpu.CompilerParams(needs_layout_passes=False),
        out_type=jax.ShapeDtypeStruct((SAMPLE_NUM,), jnp.int32),
        scratch_types=[
            pltpu.VMEM((SAMPLE_NUM,), jnp.int32),
            pltpu.VMEM((CPAD,), jnp.int32),
            pltpu.VMEM((CPAD,), jnp.int32),
        ],
    )
    def k(rp_h, rk_h, ci_h, rows_o, rows_v, rk_v, ci_v):
        wid = lax.axis_index("s") * info.num_cores + lax.axis_index("c")

        @pl.when(wid == 0)
        def _():
            pltpu.sync_copy(rp_h, rows_v)
            pltpu.sync_copy(rk_h, rk_v)
            pltpu.sync_copy(ci_h, ci_v)

            def sc(i, _):
                rv = rk_v[pl.ds(i * 16, 16)]
                cv = ci_v[pl.ds(i * 16, 16)]
                m = (rv < RNUM) & (cv >= 0)
                dst = B + jnp.minimum(rv, RNUM - 1)
                plsc.store_scatter(rows_v, [dst], cv, mask=m)
                return 0
            lax.fori_loop(0, CPAD // 16, sc, 0)
            pltpu.sync_copy(rows_v, rows_o)

    return k(rows_part, rank, candidx)


def kernel(features, labels, weight):
    bsz = features.shape[0]

    fn = features / (jnp.linalg.norm(features, axis=1, keepdims=True) + 1e-12)
    wn = weight / (jnp.linalg.norm(weight, axis=1, keepdims=True) + 1e-12)
    wn_pad = jnp.pad(wn, ((0, N_PAD - N), (0, 0)))

    scores, m1t, agg = _scores_call(fn, wn_pad)

    selT = _blocksel_call(m1t)
    sel = selT.T
    gath = _sc_gather(scores.reshape(NBLK * B, 128), sel.reshape(-1),
                      (B * KSEL, 128))
    topidx = _top9_call(gath.reshape(B, KSEL * 128), sel)

    cand, pos = _sc_masks(topidx.reshape(-1), labels)

    p2d, q2d, inv, fo = _prefix_call(cand.reshape(NBLK, 128),
                                     pos.reshape(NBLK, 128),
                                     labels.reshape(B, 1), labels)
    pflat = p2d.reshape(-1).astype(jnp.int32)
    qflat = q2d.reshape(-1).astype(jnp.int32)
    c_tot = pflat[-1]
    f_tot = jnp.maximum(RNUM - c_tot, 0)
    cvec = jnp.full((16,), B, jnp.int32) + c_tot
    fvec = jnp.full((16,), 0, jnp.int32) + f_tot

    aggc, candidx, rows_part = _sc_compact(
        cand, pos, pflat, qflat, agg, labels, fo, inv, cvec, fvec)

    rank = _rank_call(aggc, candidx).reshape(-1)

    rows = _sc_rows(rows_part, rank, candidx)

    w_sel = _sc_gather(weight, rows, (SAMPLE_NUM, FDIM))
    bias = jnp.zeros((SAMPLE_NUM,), jnp.float32)
    return w_sel, bias, inv.astype(jnp.int64)


# speedup vs baseline: 5.5491x; 1.0550x over previous
import functools

import jax
import jax.numpy as jnp
from jax import lax
from jax.experimental import pallas as pl
from jax.experimental.pallas import tpu as pltpu
from jax.experimental.pallas import tpu_sc as plsc

B = 1024
FDIM = 128
SAMPLE_NUM = 8192
N = 100000
N_PAD = 100352
TILE = 1024
NBLK = N_PAD // 128
KSEL = 16
NNBR = SAMPLE_NUM // B + 1
NEG_INF = -3.0e38


def _scores_body(fn_ref, w_ref, s_ref, m1t_ref, agg_ref):
    t = pl.program_id(0)
    s = jax.lax.dot_general(
        fn_ref[...], w_ref[...],
        (((1,), (1,)), ((), ())),
        preferred_element_type=jnp.float32,
    )
    col = t * TILE + lax.broadcasted_iota(jnp.int32, (1, TILE), 1)
    s = jnp.where(col < N, s, NEG_INF)
    s_ref[...] = jnp.stack([s[:, 128 * b:128 * (b + 1)] for b in range(8)],
                           axis=0)
    maxes = [jnp.max(s[:, 128 * b:128 * (b + 1)], axis=1) for b in range(8)]
    m1t_ref[...] = jnp.stack(maxes, axis=0)
    agg_ref[...] = jnp.max(s, axis=0)


def _scores_call(fn, weight):
    grid = N_PAD // TILE
    return pl.pallas_call(
        _scores_body,
        grid=(grid,),
        in_specs=[
            pl.BlockSpec((B, FDIM), lambda t: (0, 0)),
            pl.BlockSpec((TILE, FDIM), lambda t: (t, 0)),
        ],
        out_specs=[
            pl.BlockSpec((8, B, 128), lambda t: (t, 0, 0)),
            pl.BlockSpec((8, B), lambda t: (t, 0)),
            pl.BlockSpec((TILE,), lambda t: (t,)),
        ],
        out_shape=[
            jax.ShapeDtypeStruct((NBLK, B, 128), jnp.float32),
            jax.ShapeDtypeStruct((NBLK, B), jnp.float32),
            jax.ShapeDtypeStruct((N_PAD,), jnp.float32),
        ],
    )(fn, weight)


def _blocksel_body(m1t_ref, sel_ref):
    v = m1t_ref[...]
    blk = lax.broadcasted_iota(jnp.int32, (NBLK, B), 0)
    rows = []
    for _ in range(KSEL):
        m = jnp.max(v, axis=0, keepdims=True)
        bi = jnp.min(jnp.where(v == m, blk, NBLK), axis=0, keepdims=True)
        rows.append(bi)
        v = jnp.where(blk == bi, NEG_INF, v)
    q = lax.broadcasted_iota(jnp.int32, (KSEL, B), 1)
    sel_ref[...] = jnp.concatenate(rows, axis=0) * B + q


def _blocksel_call(m1t):
    return pl.pallas_call(
        _blocksel_body,
        in_specs=[pl.BlockSpec((NBLK, B), lambda: (0, 0))],
        out_specs=pl.BlockSpec((KSEL, B), lambda: (0, 0)),
        out_shape=jax.ShapeDtypeStruct((KSEL, B), jnp.int32),
    )(m1t)


def _sc_gather(table, idx, rows_out_shape):
    info = plsc.get_sparse_core_info()
    nw = info.num_cores * info.num_subcores
    n_idx = idx.shape[0]
    d = table.shape[1]
    b_per_w = n_idx // nw
    mesh = plsc.VectorSubcoreMesh(core_axis_name="c", subcore_axis_name="s")

    @functools.partial(
        pl.kernel, mesh=mesh,
        out_type=jax.ShapeDtypeStruct(rows_out_shape, table.dtype),
        scratch_types=[
            pltpu.VMEM((b_per_w,), jnp.int32),
            pltpu.VMEM((b_per_w, d), table.dtype),
            pltpu.SemaphoreType.DMA,
        ],
    )
    def k(table_hbm, idx_hbm, out_hbm, idx_v, rows_v, sem):
        wid = lax.axis_index("s") * info.num_cores + lax.axis_index("c")
        base = wid * b_per_w
        pltpu.sync_copy(idx_hbm.at[pl.ds(base, b_per_w)], idx_v)
        pltpu.async_copy(table_hbm.at[idx_v], rows_v, sem).wait()
        pltpu.sync_copy(rows_v, out_hbm.at[pl.ds(base, b_per_w)])

    return k(table, idx)


def _top9_body(g_ref, bsel_ref, out_ref):
    v = g_ref[...]
    lane = lax.broadcasted_iota(jnp.int32, (1, 128), 1)
    chunks = []
    for k in range(KSEL):
        b = bsel_ref[:, k:k + 1] // B
        chunks.append(b * 128 + lane)
    gidx = jnp.concatenate(chunks, axis=1)
    outs = []
    for r in range(NNBR):
        m = jnp.max(v, axis=1, keepdims=True)
        g = jnp.min(jnp.where(v == m, gidx, N_PAD), axis=1, keepdims=True)
        outs.append(g)
        v = jnp.where(gidx == g, NEG_INF, v)
    outs += [outs[0]] * (KSEL - NNBR)
    out_ref[...] = jnp.concatenate(outs, axis=1)


def _top9_call(gath2d, bsel):
    return pl.pallas_call(
        _top9_body,
        in_specs=[
            pl.BlockSpec((B, KSEL * 128), lambda: (0, 0)),
            pl.BlockSpec((B, KSEL), lambda: (0, 0)),
        ],
        out_specs=pl.BlockSpec((B, KSEL), lambda: (0, 0)),
        out_shape=jax.ShapeDtypeStruct((B, KSEL), jnp.int32),
    )(gath2d, bsel)


RNUM = SAMPLE_NUM - B
NCAND = KSEL * B
CPAD = 9216


def _prefix_body(cand_ref, pos_ref, lab2_ref, lab_ref, p_ref, q_ref,
                 inv_ref, fo_ref):
    cand = cand_ref[...]
    pos = pos_ref[...]
    candnp = cand * (1.0 - pos)
    jr = lax.broadcasted_iota(jnp.int32, (NBLK, 128), 0)
    jc = lax.broadcasted_iota(jnp.int32, (NBLK, 128), 1)
    valid = (jr * 128 + jc) < N
    fb = jnp.where(valid, (1.0 - cand) * (1.0 - pos), 0.0)

    ia = lax.broadcasted_iota(jnp.int32, (NBLK, NBLK), 0)
    ib = lax.broadcasted_iota(jnp.int32, (NBLK, NBLK), 1)
    slt = (ia > ib).astype(jnp.float32)
    ua = lax.broadcasted_iota(jnp.int32, (128, 128), 0)
    ub = lax.broadcasted_iota(jnp.int32, (128, 128), 1)
    su = (ua < ub).astype(jnp.float32)
    ones = jnp.ones((128, 128), jnp.float32)

    def mm(a, b):
        return jax.lax.dot_general(a, b, (((1,), (0,)), ((), ())),
                                   preferred_element_type=jnp.float32)

    p_ref[...] = mm(slt, mm(candnp, ones)) + mm(candnp, su)
    q_ref[...] = mm(slt, mm(fb, ones)) + mm(fb, su)

    la = lab2_ref[...]
    lb = lab_ref[...].reshape(1, B)
    pa = lax.broadcasted_iota(jnp.int32, (B, B), 0)
    pb = lax.broadcasted_iota(jnp.int32, (B, B), 1)
    earlier = ((la == lb) & (pa < pb)).astype(jnp.float32)
    fo = jnp.sum(earlier, axis=0) == 0.0
    lt = (la < lb).astype(jnp.float32)
    fo8 = jnp.broadcast_to(fo.astype(jnp.float32).reshape(1, B), (8, B))
    invm = mm(fo8, lt)
    inv_ref[...] = invm[0].astype(jnp.int32)
    fo_ref[...] = fo.astype(jnp.int32)


def _prefix_call(cand2d, pos2d, labels2d, labels):
    return pl.pallas_call(
        _prefix_body,
        in_specs=[
            pl.BlockSpec((NBLK, 128), lambda: (0, 0)),
            pl.BlockSpec((NBLK, 128), lambda: (0, 0)),
            pl.BlockSpec((B, 1), lambda: (0, 0)),
            pl.BlockSpec((B,), lambda: (0,)),
        ],
        out_specs=[
            pl.BlockSpec((NBLK, 128), lambda: (0, 0)),
            pl.BlockSpec((NBLK, 128), lambda: (0, 0)),
            pl.BlockSpec((B,), lambda: (0,)),
            pl.BlockSpec((B,), lambda: (0,)),
        ],
        out_shape=[
            jax.ShapeDtypeStruct((NBLK, 128), jnp.float32),
            jax.ShapeDtypeStruct((NBLK, 128), jnp.float32),
            jax.ShapeDtypeStruct((B,), jnp.int32),
            jax.ShapeDtypeStruct((B,), jnp.int32),
        ],
    )(cand2d, pos2d, labels2d, labels)


GI = 512


def _rank_body(a2_ref, i2_ref, af_ref, if_ref, r_ref):
    ai = a2_ref[...]
    ii = i2_ref[...]
    aj = af_ref[...].reshape(1, CPAD)
    ij = if_ref[...].reshape(1, CPAD)
    gt = (aj > ai) | ((aj == ai) & (ij < ii))
    r_ref[...] = jnp.sum(gt.astype(jnp.float32), axis=1,
                         keepdims=True).astype(jnp.int32)


def _rank_call(aggc, candidx):
    return pl.pallas_call(
        _rank_body,
        grid=(CPAD // GI,),
        in_specs=[
            pl.BlockSpec((GI, 1), lambda t: (t, 0)),
            pl.BlockSpec((GI, 1), lambda t: (t, 0)),
            pl.BlockSpec((CPAD,), lambda t: (0,)),
            pl.BlockSpec((CPAD,), lambda t: (0,)),
        ],
        out_specs=pl.BlockSpec((GI, 1), lambda t: (t, 0)),
        out_shape=jax.ShapeDtypeStruct((CPAD, 1), jnp.int32),
    )(aggc.reshape(CPAD, 1), candidx.reshape(CPAD, 1), aggc, candidx)


def _sc_masks(topidx_flat, labels):
    mesh = plsc.VectorSubcoreMesh(core_axis_name="c", subcore_axis_name="s")
    info = plsc.get_sparse_core_info()

    @functools.partial(
        pl.kernel, mesh=mesh,
        compiler_params=pltpu.CompilerParams(needs_layout_passes=False),
        out_type=[
            jax.ShapeDtypeStruct((N_PAD,), jnp.float32),
            jax.ShapeDtypeStruct((N_PAD,), jnp.float32),
        ],
        scratch_types=[
            pltpu.VMEM((N_PAD,), jnp.float32),
            pltpu.VMEM((NCAND,), jnp.int32),
        ],
    )
    def k(ti_hbm, lab_hbm, cand_hbm, pos_hbm, mask_v, idx_v):
        wid = lax.axis_index("s") * info.num_cores + lax.axis_index("c")
        ones16 = jnp.ones((16,), jnp.float32)

        @pl.when(wid == 0)
        def _():
            def zf(i, _):
                mask_v[pl.ds(i * 16, 16)] = jnp.zeros((16,), jnp.float32)
                return 0
            lax.fori_loop(0, N_PAD // 16, zf, 0)
            pltpu.sync_copy(ti_hbm, idx_v)

            def sc(i, _):
                iv = idx_v[pl.ds(i * 16, 16)]
                plsc.store_scatter(mask_v, [iv], ones16)
                return 0
            lax.fori_loop(0, NCAND // 16, sc, 0)
            pltpu.sync_copy(mask_v, cand_hbm)

        @pl.when(wid == 1)
        def _():
            def zf(i, _):
                mask_v[pl.ds(i * 16, 16)] = jnp.zeros((16,), jnp.float32)
                return 0
            lax.fori_loop(0, N_PAD // 16, zf, 0)
            pltpu.sync_copy(lab_hbm, idx_v.at[pl.ds(0, B)])

            def sc(i, _):
                iv = idx_v[pl.ds(i * 16, 16)]
                plsc.store_scatter(mask_v, [iv], ones16)
                return 0
            lax.fori_loop(0, B // 16, sc, 0)
            pltpu.sync_copy(mask_v, pos_hbm)

    return k(topidx_flat, labels)


FCH = 14336
NCHUNK = N_PAD // FCH


def _sc_compact(cand, pos, pflat, qflat, agg, labels, fo, inv, cvec, fvec):
    mesh = plsc.VectorSubcoreMesh(core_axis_name="c", subcore_axis_name="s")
    info = plsc.get_sparse_core_info()

    @functools.partial(
        pl.kernel, mesh=mesh,
        compiler_params=pltpu.CompilerParams(needs_layout_passes=False),
        out_type=[
            jax.ShapeDtypeStruct((CPAD,), jnp.float32),
            jax.ShapeDtypeStruct((CPAD,), jnp.int32),
            jax.ShapeDtypeStruct((SAMPLE_NUM,), jnp.int32),
        ],
        scratch_types=[
            pltpu.VMEM((CPAD,), jnp.float32),
            pltpu.VMEM((CPAD,), jnp.int32),
            pltpu.VMEM((SAMPLE_NUM,), jnp.int32),
            pltpu.VMEM((FCH,), jnp.float32),
            pltpu.VMEM((FCH,), jnp.float32),
            pltpu.VMEM((FCH,), jnp.int32),
            pltpu.VMEM((FCH,), jnp.float32),
            pltpu.VMEM((B,), jnp.int32),
            pltpu.VMEM((B,), jnp.int32),
            pltpu.VMEM((B,), jnp.int32),
            pltpu.VMEM((16,), jnp.int32),
            pltpu.VMEM((16,), jnp.int32),
            pltpu.SemaphoreType.DMA,
        ],
    )
    def k(cand_h, pos_h, p_h, q_h, agg_h, lab_h, fo_h, inv_h, cv_h, fv_h,
          aggc_o, candidx_o, rows_o,
          aggc_v, cidx_v, rows_v, cc_v, pc_v, pq_v, ac_v,
          lab_v, fo_v, inv_v, cvec_v, fvec_v, sem):
        wid = lax.axis_index("s") * info.num_cores + lax.axis_index("c")

        @pl.when(wid == 0)
        def _():
            def init(i, _):
                aggc_v[pl.ds(i * 16, 16)] = jnp.full((16,), NEG_INF, jnp.float32)
                cidx_v[pl.ds(i * 16, 16)] = jnp.full((16,), -1, jnp.int32)
                return 0
            lax.fori_loop(0, CPAD // 16, init, 0)

            for c in range(NCHUNK):
                sl = pl.ds(c * FCH, FCH)
                h1 = pltpu.async_copy(cand_h.at[sl], cc_v, sem)
                h2 = pltpu.async_copy(pos_h.at[sl], pc_v, sem)
                h3 = pltpu.async_copy(p_h.at[sl], pq_v, sem)
                h4 = pltpu.async_copy(agg_h.at[sl], ac_v, sem)
                h1.wait(); h2.wait(); h3.wait(); h4.wait()
                base = c * FCH

                def sweep(i, _):
                    jv = lax.iota(jnp.int32, 16) + (base + i * 16)
                    cv = cc_v[pl.ds(i * 16, 16)]
                    pv = pc_v[pl.ds(i * 16, 16)]
                    Pv = pq_v[pl.ds(i * 16, 16)]
                    av = ac_v[pl.ds(i * 16, 16)]
                    isc = (cv > 0.0) & (pv == 0.0)
                    Pc = jnp.minimum(Pv, CPAD - 1)
                    plsc.store_scatter(aggc_v, [Pc], av, mask=isc)
                    plsc.store_scatter(cidx_v, [Pc], jv, mask=isc)
                    return 0
                lax.fori_loop(0, FCH // 16, sweep, 0)

            pltpu.sync_copy(aggc_v, aggc_o)
            pltpu.sync_copy(cidx_v, candidx_o)

        @pl.when(wid == 1)
        def _():
            def zrow(i, _):
                rows_v[pl.ds(i * 16, 16)] = jnp.zeros((16,), jnp.int32)
                return 0
            lax.fori_loop(0, B // 16, zrow, 0)

            pltpu.sync_copy(lab_h, lab_v)
            pltpu.sync_copy(fo_h, fo_v)
            pltpu.sync_copy(inv_h, inv_v)
            pltpu.sync_copy(cv_h, cvec_v)
            pltpu.sync_copy(fv_h, fvec_v)

            def usc(i, _):
                lab = lab_v[pl.ds(i * 16, 16)]
                dst = inv_v[pl.ds(i * 16, 16)]
                m = fo_v[pl.ds(i * 16, 16)] != 0
                plsc.store_scatter(rows_v, [jnp.minimum(dst, B - 1)], lab,
                                   mask=m)
                return 0
            lax.fori_loop(0, B // 16, usc, 0)

            cvec = cvec_v[pl.ds(0, 16)]
            fvec = fvec_v[pl.ds(0, 16)]
            for c in range(NCHUNK):
                sl = pl.ds(c * FCH, FCH)
                h1 = pltpu.async_copy(cand_h.at[sl], cc_v, sem)
                h2 = pltpu.async_copy(pos_h.at[sl], pc_v, sem)
                h3 = pltpu.async_copy(q_h.at[sl], pq_v, sem)
                h1.wait(); h2.wait(); h3.wait()
                base = c * FCH

                def fsw(i, _):
                    jv = lax.iota(jnp.int32, 16) + (base + i * 16)
                    cv = cc_v[pl.ds(i * 16, 16)]
                    pv = pc_v[pl.ds(i * 16, 16)]
                    Qv = pq_v[pl.ds(i * 16, 16)]
                    fbm = ((cv == 0.0) & (pv == 0.0) & (jv < N)
                           & (Qv < fvec))
                    dst = jnp.minimum(cvec + Qv, SAMPLE_NUM - 1)
                    plsc.store_scatter(rows_v, [dst], jv, mask=fbm)
                    return 0
                lax.fori_loop(0, FCH // 16, fsw, 0)

            pltpu.sync_copy(rows_v, rows_o)

    return k(cand, pos, pflat, qflat, agg, labels, fo, inv, cvec, fvec)


def _sc_rows(rows_part, rank, candidx):
    mesh = plsc.VectorSubcoreMesh(core_axis_name="c", subcore_axis_name="s")
    info = plsc.get_sparse_core_info()

    @functools.partial(
        pl.kernel, mesh=mesh,
        compiler_params=pltpu.CompilerParams(needs_layout_passes=False),
        out_type=jax.ShapeDtypeStruct((SAMPLE_NUM,), jnp.int32),
        scratch_types=[
            pltpu.VMEM((SAMPLE_NUM,), jnp.int32),
            pltpu.VMEM((CPAD,), jnp.int32),
            pltpu.VMEM((CPAD,), jnp.int32),
        ],
    )
    def k(rp_h, rk_h, ci_h, rows_o, rows_v, rk_v, ci_v):
        wid = lax.axis_index("s") * info.num_cores + lax.axis_index("c")

        @pl.when(wid == 0)
        def _():
            pltpu.sync_copy(rp_h, rows_v)
            pltpu.sync_copy(rk_h, rk_v)
            pltpu.sync_copy(ci_h, ci_v)

            def sc(i, _):
                rv = rk_v[pl.ds(i * 16, 16)]
                cv = ci_v[pl.ds(i * 16, 16)]
                m = (rv < RNUM) & (cv >= 0)
                dst = B + jnp.minimum(rv, RNUM - 1)
                plsc.store_scatter(rows_v, [dst], cv, mask=m)
                return 0
            lax.fori_loop(0, CPAD // 16, sc, 0)
            pltpu.sync_copy(rows_v, rows_o)

    return k(rows_part, rank, candidx)


def kernel(features, labels, weight):
    bsz = features.shape[0]

    fn = features / (jnp.linalg.norm(features, axis=1, keepdims=True) + 1e-12)
    wn = weight / (jnp.linalg.norm(weight, axis=1, keepdims=True) + 1e-12)

    scores, m1t, agg = _scores_call(fn, wn)

    selT = _blocksel_call(m1t)
    sel = selT.T
    gath = _sc_gather(scores.reshape(NBLK * B, 128), sel.reshape(-1),
                      (B * KSEL, 128))
    topidx = _top9_call(gath.reshape(B, KSEL * 128), sel)

    cand, pos = _sc_masks(topidx.reshape(-1), labels)

    p2d, q2d, inv, fo = _prefix_call(cand.reshape(NBLK, 128),
                                     pos.reshape(NBLK, 128),
                                     labels.reshape(B, 1), labels)
    pflat = p2d.reshape(-1).astype(jnp.int32)
    qflat = q2d.reshape(-1).astype(jnp.int32)
    c_tot = pflat[-1]
    f_tot = jnp.maximum(RNUM - c_tot, 0)
    cvec = jnp.full((16,), B, jnp.int32) + c_tot
    fvec = jnp.full((16,), 0, jnp.int32) + f_tot

    aggc, candidx, rows_part = _sc_compact(
        cand, pos, pflat, qflat, agg, labels, fo, inv, cvec, fvec)

    rank = _rank_call(aggc, candidx).reshape(-1)

    rows = _sc_rows(rows_part, rank, candidx)

    w_sel = _sc_gather(weight, rows, (SAMPLE_NUM, FDIM))
    bias = jnp.zeros((SAMPLE_NUM,), jnp.float32)
    return w_sel, bias, inv.astype(jnp.int64)


# speedup vs baseline: 5.7019x; 1.0275x over previous
import functools

import jax
import jax.numpy as jnp
from jax import lax
from jax.experimental import pallas as pl
from jax.experimental.pallas import tpu as pltpu
from jax.experimental.pallas import tpu_sc as plsc

B = 1024
FDIM = 128
SAMPLE_NUM = 8192
N = 100000
N_PAD = 100352
TILE = 2048
NBLK = N_PAD // 128
KSEL = 16
NNBR = SAMPLE_NUM // B + 1
NEG_INF = -3.0e38


NBT = TILE // 128


def _scores_body(fn_ref, w_ref, d_ref, s_ref, m1t_ref, agg_ref):
    t = pl.program_id(0)
    wn = w_ref[...] / d_ref[...]
    s = jax.lax.dot_general(
        fn_ref[...], wn,
        (((1,), (1,)), ((), ())),
        preferred_element_type=jnp.float32,
    )
    col = t * TILE + lax.broadcasted_iota(jnp.int32, (1, TILE), 1)
    s = jnp.where(col < N, s, NEG_INF)
    s_ref[...] = jnp.stack([s[:, 128 * b:128 * (b + 1)] for b in range(NBT)],
                           axis=0)
    maxes = [jnp.max(s[:, 128 * b:128 * (b + 1)], axis=1) for b in range(NBT)]
    m1t_ref[...] = jnp.stack(maxes, axis=0)
    agg_ref[...] = jnp.max(s, axis=0)


def _scores_call(fn, weight, denom):
    grid = N_PAD // TILE
    return pl.pallas_call(
        _scores_body,
        grid=(grid,),
        in_specs=[
            pl.BlockSpec((B, FDIM), lambda t: (0, 0)),
            pl.BlockSpec((TILE, FDIM), lambda t: (t, 0)),
            pl.BlockSpec((TILE, 1), lambda t: (t, 0)),
        ],
        out_specs=[
            pl.BlockSpec((NBT, B, 128), lambda t: (t, 0, 0)),
            pl.BlockSpec((NBT, B), lambda t: (t, 0)),
            pl.BlockSpec((TILE,), lambda t: (t,)),
        ],
        out_shape=[
            jax.ShapeDtypeStruct((NBLK, B, 128), jnp.float32),
            jax.ShapeDtypeStruct((NBLK, B), jnp.float32),
            jax.ShapeDtypeStruct((N_PAD,), jnp.float32),
        ],
    )(fn, weight, denom)


def _blocksel_body(m1t_ref, sel_ref):
    v = m1t_ref[...]
    blk = lax.broadcasted_iota(jnp.int32, (NBLK, B), 0)
    rows = []
    for _ in range(KSEL):
        m = jnp.max(v, axis=0, keepdims=True)
        bi = jnp.min(jnp.where(v == m, blk, NBLK), axis=0, keepdims=True)
        rows.append(bi)
        v = jnp.where(blk == bi, NEG_INF, v)
    q = lax.broadcasted_iota(jnp.int32, (KSEL, B), 1)
    sel_ref[...] = jnp.concatenate(rows, axis=0) * B + q


def _blocksel_call(m1t):
    return pl.pallas_call(
        _blocksel_body,
        in_specs=[pl.BlockSpec((NBLK, B), lambda: (0, 0))],
        out_specs=pl.BlockSpec((KSEL, B), lambda: (0, 0)),
        out_shape=jax.ShapeDtypeStruct((KSEL, B), jnp.int32),
    )(m1t)


def _sc_gather(table, idx, rows_out_shape):
    info = plsc.get_sparse_core_info()
    nw = info.num_cores * info.num_subcores
    n_idx = idx.shape[0]
    d = table.shape[1]
    b_per_w = n_idx // nw
    mesh = plsc.VectorSubcoreMesh(core_axis_name="c", subcore_axis_name="s")

    @functools.partial(
        pl.kernel, mesh=mesh,
        out_type=jax.ShapeDtypeStruct(rows_out_shape, table.dtype),
        scratch_types=[
            pltpu.VMEM((b_per_w,), jnp.int32),
            pltpu.VMEM((b_per_w, d), table.dtype),
            pltpu.SemaphoreType.DMA,
        ],
    )
    def k(table_hbm, idx_hbm, out_hbm, idx_v, rows_v, sem):
        wid = lax.axis_index("s") * info.num_cores + lax.axis_index("c")
        base = wid * b_per_w
        pltpu.sync_copy(idx_hbm.at[pl.ds(base, b_per_w)], idx_v)
        pltpu.async_copy(table_hbm.at[idx_v], rows_v, sem).wait()
        pltpu.sync_copy(rows_v, out_hbm.at[pl.ds(base, b_per_w)])

    return k(table, idx)


def _top9_body(g_ref, bsel_ref, out_ref):
    v = g_ref[...]
    lane = lax.broadcasted_iota(jnp.int32, (1, 128), 1)
    chunks = []
    for k in range(KSEL):
        b = bsel_ref[:, k:k + 1] // B
        chunks.append(b * 128 + lane)
    gidx = jnp.concatenate(chunks, axis=1)
    outs = []
    for r in range(NNBR):
        m = jnp.max(v, axis=1, keepdims=True)
        g = jnp.min(jnp.where(v == m, gidx, N_PAD), axis=1, keepdims=True)
        outs.append(g)
        v = jnp.where(gidx == g, NEG_INF, v)
    outs += [outs[0]] * (KSEL - NNBR)
    out_ref[...] = jnp.concatenate(outs, axis=1)


def _top9_call(gath2d, bsel):
    return pl.pallas_call(
        _top9_body,
        in_specs=[
            pl.BlockSpec((B, KSEL * 128), lambda: (0, 0)),
            pl.BlockSpec((B, KSEL), lambda: (0, 0)),
        ],
        out_specs=pl.BlockSpec((B, KSEL), lambda: (0, 0)),
        out_shape=jax.ShapeDtypeStruct((B, KSEL), jnp.int32),
    )(gath2d, bsel)


RNUM = SAMPLE_NUM - B
NCAND = KSEL * B
CPAD = 9216


def _prefix_body(cand_ref, pos_ref, lab2_ref, lab_ref, p_ref, q_ref,
                 inv_ref, fo_ref):
    cand = cand_ref[...]
    pos = pos_ref[...]
    candnp = cand * (1.0 - pos)
    jr = lax.broadcasted_iota(jnp.int32, (NBLK, 128), 0)
    jc = lax.broadcasted_iota(jnp.int32, (NBLK, 128), 1)
    valid = (jr * 128 + jc) < N
    fb = jnp.where(valid, (1.0 - cand) * (1.0 - pos), 0.0)

    ia = lax.broadcasted_iota(jnp.int32, (NBLK, NBLK), 0)
    ib = lax.broadcasted_iota(jnp.int32, (NBLK, NBLK), 1)
    slt = (ia > ib).astype(jnp.float32)
    ua = lax.broadcasted_iota(jnp.int32, (128, 128), 0)
    ub = lax.broadcasted_iota(jnp.int32, (128, 128), 1)
    su = (ua < ub).astype(jnp.float32)
    ones = jnp.ones((128, 128), jnp.float32)

    def mm(a, b):
        return jax.lax.dot_general(a, b, (((1,), (0,)), ((), ())),
                                   preferred_element_type=jnp.float32)

    p_ref[...] = mm(slt, mm(candnp, ones)) + mm(candnp, su)
    q_ref[...] = mm(slt, mm(fb, ones)) + mm(fb, su)

    la = lab2_ref[...]
    lb = lab_ref[...].reshape(1, B)
    pa = lax.broadcasted_iota(jnp.int32, (B, B), 0)
    pb = lax.broadcasted_iota(jnp.int32, (B, B), 1)
    earlier = ((la == lb) & (pa < pb)).astype(jnp.float32)
    fo = jnp.sum(earlier, axis=0) == 0.0
    lt = (la < lb).astype(jnp.float32)
    fo8 = jnp.broadcast_to(fo.astype(jnp.float32).reshape(1, B), (8, B))
    invm = mm(fo8, lt)
    inv_ref[...] = invm[0].astype(jnp.int32)
    fo_ref[...] = fo.astype(jnp.int32)


def _prefix_call(cand2d, pos2d, labels2d, labels):
    return pl.pallas_call(
        _prefix_body,
        in_specs=[
            pl.BlockSpec((NBLK, 128), lambda: (0, 0)),
            pl.BlockSpec((NBLK, 128), lambda: (0, 0)),
            pl.BlockSpec((B, 1), lambda: (0, 0)),
            pl.BlockSpec((B,), lambda: (0,)),
        ],
        out_specs=[
            pl.BlockSpec((NBLK, 128), lambda: (0, 0)),
            pl.BlockSpec((NBLK, 128), lambda: (0, 0)),
            pl.BlockSpec((B,), lambda: (0,)),
            pl.BlockSpec((B,), lambda: (0,)),
        ],
        out_shape=[
            jax.ShapeDtypeStruct((NBLK, 128), jnp.float32),
            jax.ShapeDtypeStruct((NBLK, 128), jnp.float32),
            jax.ShapeDtypeStruct((B,), jnp.int32),
            jax.ShapeDtypeStruct((B,), jnp.int32),
        ],
    )(cand2d, pos2d, labels2d, labels)


GI = 512


def _rank_body(a2_ref, i2_ref, af_ref, if_ref, r_ref):
    ai = a2_ref[...]
    ii = i2_ref[...]
    aj = af_ref[...].reshape(1, CPAD)
    ij = if_ref[...].reshape(1, CPAD)
    gt = (aj > ai) | ((aj == ai) & (ij < ii))
    r_ref[...] = jnp.sum(gt.astype(jnp.float32), axis=1,
                         keepdims=True).astype(jnp.int32)


def _rank_call(aggc, candidx):
    return pl.pallas_call(
        _rank_body,
        grid=(CPAD // GI,),
        in_specs=[
            pl.BlockSpec((GI, 1), lambda t: (t, 0)),
            pl.BlockSpec((GI, 1), lambda t: (t, 0)),
            pl.BlockSpec((CPAD,), lambda t: (0,)),
            pl.BlockSpec((CPAD,), lambda t: (0,)),
        ],
        out_specs=pl.BlockSpec((GI, 1), lambda t: (t, 0)),
        out_shape=jax.ShapeDtypeStruct((CPAD, 1), jnp.int32),
    )(aggc.reshape(CPAD, 1), candidx.reshape(CPAD, 1), aggc, candidx)


def _sc_masks(topidx_flat, labels):
    mesh = plsc.VectorSubcoreMesh(core_axis_name="c", subcore_axis_name="s")
    info = plsc.get_sparse_core_info()

    @functools.partial(
        pl.kernel, mesh=mesh,
        compiler_params=pltpu.CompilerParams(needs_layout_passes=False),
        out_type=[
            jax.ShapeDtypeStruct((N_PAD,), jnp.float32),
            jax.ShapeDtypeStruct((N_PAD,), jnp.float32),
        ],
        scratch_types=[
            pltpu.VMEM((N_PAD,), jnp.float32),
            pltpu.VMEM((NCAND,), jnp.int32),
        ],
    )
    def k(ti_hbm, lab_hbm, cand_hbm, pos_hbm, mask_v, idx_v):
        wid = lax.axis_index("s") * info.num_cores + lax.axis_index("c")
        ones16 = jnp.ones((16,), jnp.float32)

        @pl.when(wid == 0)
        def _():
            def zf(i, _):
                mask_v[pl.ds(i * 16, 16)] = jnp.zeros((16,), jnp.float32)
                return 0
            lax.fori_loop(0, N_PAD // 16, zf, 0)
            pltpu.sync_copy(ti_hbm, idx_v)

            def sc(i, _):
                iv = idx_v[pl.ds(i * 16, 16)]
                plsc.store_scatter(mask_v, [iv], ones16)
                return 0
            lax.fori_loop(0, NCAND // 16, sc, 0)
            pltpu.sync_copy(mask_v, cand_hbm)

        @pl.when(wid == 1)
        def _():
            def zf(i, _):
                mask_v[pl.ds(i * 16, 16)] = jnp.zeros((16,), jnp.float32)
                return 0
            lax.fori_loop(0, N_PAD // 16, zf, 0)
            pltpu.sync_copy(lab_hbm, idx_v.at[pl.ds(0, B)])

            def sc(i, _):
                iv = idx_v[pl.ds(i * 16, 16)]
                plsc.store_scatter(mask_v, [iv], ones16)
                return 0
            lax.fori_loop(0, B // 16, sc, 0)
            pltpu.sync_copy(mask_v, pos_hbm)

    return k(topidx_flat, labels)


FCH = 14336
NCHUNK = N_PAD // FCH


def _sc_compact(cand, pos, pflat, qflat, agg, labels, fo, inv, cvec, fvec):
    mesh = plsc.VectorSubcoreMesh(core_axis_name="c", subcore_axis_name="s")
    info = plsc.get_sparse_core_info()

    @functools.partial(
        pl.kernel, mesh=mesh,
        compiler_params=pltpu.CompilerParams(needs_layout_passes=False),
        out_type=[
            jax.ShapeDtypeStruct((CPAD,), jnp.float32),
            jax.ShapeDtypeStruct((CPAD,), jnp.int32),
            jax.ShapeDtypeStruct((SAMPLE_NUM,), jnp.int32),
        ],
        scratch_types=[
            pltpu.VMEM((CPAD,), jnp.float32),
            pltpu.VMEM((CPAD,), jnp.int32),
            pltpu.VMEM((SAMPLE_NUM,), jnp.int32),
            pltpu.VMEM((FCH,), jnp.float32),
            pltpu.VMEM((FCH,), jnp.float32),
            pltpu.VMEM((FCH,), jnp.int32),
            pltpu.VMEM((FCH,), jnp.float32),
            pltpu.VMEM((B,), jnp.int32),
            pltpu.VMEM((B,), jnp.int32),
            pltpu.VMEM((B,), jnp.int32),
            pltpu.VMEM((16,), jnp.int32),
            pltpu.VMEM((16,), jnp.int32),
            pltpu.SemaphoreType.DMA,
        ],
    )
    def k(cand_h, pos_h, p_h, q_h, agg_h, lab_h, fo_h, inv_h, cv_h, fv_h,
          aggc_o, candidx_o, rows_o,
          aggc_v, cidx_v, rows_v, cc_v, pc_v, pq_v, ac_v,
          lab_v, fo_v, inv_v, cvec_v, fvec_v, sem):
        wid = lax.axis_index("s") * info.num_cores + lax.axis_index("c")

        @pl.when(wid == 0)
        def _():
            def init(i, _):
                aggc_v[pl.ds(i * 16, 16)] = jnp.full((16,), NEG_INF, jnp.float32)
                cidx_v[pl.ds(i * 16, 16)] = jnp.full((16,), -1, jnp.int32)
                return 0
            lax.fori_loop(0, CPAD // 16, init, 0)

            for c in range(NCHUNK):
                sl = pl.ds(c * FCH, FCH)
                h1 = pltpu.async_copy(cand_h.at[sl], cc_v, sem)
                h2 = pltpu.async_copy(pos_h.at[sl], pc_v, sem)
                h3 = pltpu.async_copy(p_h.at[sl], pq_v, sem)
                h4 = pltpu.async_copy(agg_h.at[sl], ac_v, sem)
                h1.wait(); h2.wait(); h3.wait(); h4.wait()
                base = c * FCH

                def sweep(i, _):
                    jv = lax.iota(jnp.int32, 16) + (base + i * 16)
                    cv = cc_v[pl.ds(i * 16, 16)]
                    pv = pc_v[pl.ds(i * 16, 16)]
                    Pv = pq_v[pl.ds(i * 16, 16)]
                    av = ac_v[pl.ds(i * 16, 16)]
                    isc = (cv > 0.0) & (pv == 0.0)
                    Pc = jnp.minimum(Pv, CPAD - 1)
                    plsc.store_scatter(aggc_v, [Pc], av, mask=isc)
                    plsc.store_scatter(cidx_v, [Pc], jv, mask=isc)
                    return 0
                lax.fori_loop(0, FCH // 16, sweep, 0)

            pltpu.sync_copy(aggc_v, aggc_o)
            pltpu.sync_copy(cidx_v, candidx_o)

        @pl.when(wid == 1)
        def _():
            def zrow(i, _):
                rows_v[pl.ds(i * 16, 16)] = jnp.zeros((16,), jnp.int32)
                return 0
            lax.fori_loop(0, B // 16, zrow, 0)

            pltpu.sync_copy(lab_h, lab_v)
            pltpu.sync_copy(fo_h, fo_v)
            pltpu.sync_copy(inv_h, inv_v)
            pltpu.sync_copy(cv_h, cvec_v)
            pltpu.sync_copy(fv_h, fvec_v)

            def usc(i, _):
                lab = lab_v[pl.ds(i * 16, 16)]
                dst = inv_v[pl.ds(i * 16, 16)]
                m = fo_v[pl.ds(i * 16, 16)] != 0
                plsc.store_scatter(rows_v, [jnp.minimum(dst, B - 1)], lab,
                                   mask=m)
                return 0
            lax.fori_loop(0, B // 16, usc, 0)

            cvec = cvec_v[pl.ds(0, 16)]
            fvec = fvec_v[pl.ds(0, 16)]
            for c in range(NCHUNK):
                sl = pl.ds(c * FCH, FCH)
                h1 = pltpu.async_copy(cand_h.at[sl], cc_v, sem)
                h2 = pltpu.async_copy(pos_h.at[sl], pc_v, sem)
                h3 = pltpu.async_copy(q_h.at[sl], pq_v, sem)
                h1.wait(); h2.wait(); h3.wait()
                base = c * FCH

                def fsw(i, _):
                    jv = lax.iota(jnp.int32, 16) + (base + i * 16)
                    cv = cc_v[pl.ds(i * 16, 16)]
                    pv = pc_v[pl.ds(i * 16, 16)]
                    Qv = pq_v[pl.ds(i * 16, 16)]
                    fbm = ((cv == 0.0) & (pv == 0.0) & (jv < N)
                           & (Qv < fvec))
                    dst = jnp.minimum(cvec + Qv, SAMPLE_NUM - 1)
                    plsc.store_scatter(rows_v, [dst], jv, mask=fbm)
                    return 0
                lax.fori_loop(0, FCH // 16, fsw, 0)

            pltpu.sync_copy(rows_v, rows_o)

    return k(cand, pos, pflat, qflat, agg, labels, fo, inv, cvec, fvec)


def _sc_rows(rows_part, rank, candidx):
    mesh = plsc.VectorSubcoreMesh(core_axis_name="c", subcore_axis_name="s")
    info = plsc.get_sparse_core_info()

    @functools.partial(
        pl.kernel, mesh=mesh,
        compiler_params=pltpu.CompilerParams(needs_layout_passes=False),
        out_type=jax.ShapeDtypeStruct((SAMPLE_NUM,), jnp.int32),
        scratch_types=[
            pltpu.VMEM((SAMPLE_NUM,), jnp.int32),
            pltpu.VMEM((CPAD,), jnp.int32),
            pltpu.VMEM((CPAD,), jnp.int32),
        ],
    )
    def k(rp_h, rk_h, ci_h, rows_o, rows_v, rk_v, ci_v):
        wid = lax.axis_index("s") * info.num_cores + lax.axis_index("c")

        @pl.when(wid == 0)
        def _():
            pltpu.sync_copy(rp_h, rows_v)
            pltpu.sync_copy(rk_h, rk_v)
            pltpu.sync_copy(ci_h, ci_v)

            def sc(i, _):
                rv = rk_v[pl.ds(i * 16, 16)]
                cv = ci_v[pl.ds(i * 16, 16)]
                m = (rv < RNUM) & (cv >= 0)
                dst = B + jnp.minimum(rv, RNUM - 1)
                plsc.store_scatter(rows_v, [dst], cv, mask=m)
                return 0
            lax.fori_loop(0, CPAD // 16, sc, 0)
            pltpu.sync_copy(rows_v, rows_o)

    return k(rows_part, rank, candidx)


def kernel(features, labels, weight):
    bsz = features.shape[0]

    fn = features / (jnp.linalg.norm(features, axis=1, keepdims=True) + 1e-12)
    denom = jnp.linalg.norm(weight, axis=1, keepdims=True) + 1e-12

    scores, m1t, agg = _scores_call(fn, weight, denom)

    selT = _blocksel_call(m1t)
    sel = selT.T
    gath = _sc_gather(scores.reshape(NBLK * B, 128), sel.reshape(-1),
                      (B * KSEL, 128))
    topidx = _top9_call(gath.reshape(B, KSEL * 128), sel)

    cand, pos = _sc_masks(topidx.reshape(-1), labels)

    p2d, q2d, inv, fo = _prefix_call(cand.reshape(NBLK, 128),
                                     pos.reshape(NBLK, 128),
                                     labels.reshape(B, 1), labels)
    pflat = p2d.reshape(-1).astype(jnp.int32)
    qflat = q2d.reshape(-1).astype(jnp.int32)
    c_tot = pflat[-1]
    f_tot = jnp.maximum(RNUM - c_tot, 0)
    cvec = jnp.full((16,), B, jnp.int32) + c_tot
    fvec = jnp.full((16,), 0, jnp.int32) + f_tot

    aggc, candidx, rows_part = _sc_compact(
        cand, pos, pflat, qflat, agg, labels, fo, inv, cvec, fvec)

    rank = _rank_call(aggc, candidx).reshape(-1)

    rows = _sc_rows(rows_part, rank, candidx)

    w_sel = _sc_gather(weight, rows, (SAMPLE_NUM, FDIM))
    bias = jnp.zeros((SAMPLE_NUM,), jnp.float32)
    return w_sel, bias, inv.astype(jnp.int64)


# speedup vs baseline: 6.3446x; 1.1127x over previous
import functools

import jax
import jax.numpy as jnp
from jax import lax
from jax.experimental import pallas as pl
from jax.experimental.pallas import tpu as pltpu
from jax.experimental.pallas import tpu_sc as plsc

B = 1024
FDIM = 128
SAMPLE_NUM = 8192
N = 100000
N_PAD = 100352
TILE = 2048
NBLK = N_PAD // 128
KSEL = 16
NNBR = SAMPLE_NUM // B + 1
NEG_INF = -3.0e38


NBT = TILE // 128


def _scores_body(fn_ref, w_ref, d_ref, s_ref, m1t_ref, agg_ref):
    t = pl.program_id(0)
    wn = w_ref[...] / d_ref[...].reshape(TILE, 1)
    s = jax.lax.dot_general(
        fn_ref[...], wn,
        (((1,), (1,)), ((), ())),
        preferred_element_type=jnp.float32,
    )
    col = t * TILE + lax.broadcasted_iota(jnp.int32, (1, TILE), 1)
    s = jnp.where(col < N, s, NEG_INF)
    s_ref[...] = jnp.stack([s[:, 128 * b:128 * (b + 1)] for b in range(NBT)],
                           axis=0)
    maxes = [jnp.max(s[:, 128 * b:128 * (b + 1)], axis=1) for b in range(NBT)]
    m1t_ref[...] = jnp.stack(maxes, axis=0)
    agg_ref[...] = jnp.max(s, axis=0)


def _scores_call(fn, weight, denom):
    grid = N_PAD // TILE
    return pl.pallas_call(
        _scores_body,
        grid=(grid,),
        in_specs=[
            pl.BlockSpec((B, FDIM), lambda t: (0, 0)),
            pl.BlockSpec((TILE, FDIM), lambda t: (t, 0)),
            pl.BlockSpec((TILE,), lambda t: (t,)),
        ],
        out_specs=[
            pl.BlockSpec((NBT, B, 128), lambda t: (t, 0, 0)),
            pl.BlockSpec((NBT, B), lambda t: (t, 0)),
            pl.BlockSpec((TILE,), lambda t: (t,)),
        ],
        out_shape=[
            jax.ShapeDtypeStruct((NBLK, B, 128), jnp.float32),
            jax.ShapeDtypeStruct((NBLK, B), jnp.float32),
            jax.ShapeDtypeStruct((N_PAD,), jnp.float32),
        ],
    )(fn, weight, denom)


def _blocksel_body(m1t_ref, sel_ref):
    v = m1t_ref[...]
    blk = lax.broadcasted_iota(jnp.int32, (NBLK, B), 0)
    rows = []
    for _ in range(KSEL):
        m = jnp.max(v, axis=0, keepdims=True)
        bi = jnp.min(jnp.where(v == m, blk, NBLK), axis=0, keepdims=True)
        rows.append(bi)
        v = jnp.where(blk == bi, NEG_INF, v)
    q = lax.broadcasted_iota(jnp.int32, (KSEL, B), 1)
    sel_ref[...] = jnp.concatenate(rows, axis=0) * B + q


def _blocksel_call(m1t):
    return pl.pallas_call(
        _blocksel_body,
        in_specs=[pl.BlockSpec((NBLK, B), lambda: (0, 0))],
        out_specs=pl.BlockSpec((KSEL, B), lambda: (0, 0)),
        out_shape=jax.ShapeDtypeStruct((KSEL, B), jnp.int32),
    )(m1t)


def _sc_gather(table, idx, rows_out_shape):
    info = plsc.get_sparse_core_info()
    nw = info.num_cores * info.num_subcores
    n_idx = idx.shape[0]
    d = table.shape[1]
    b_per_w = n_idx // nw
    mesh = plsc.VectorSubcoreMesh(core_axis_name="c", subcore_axis_name="s")

    @functools.partial(
        pl.kernel, mesh=mesh,
        out_type=jax.ShapeDtypeStruct(rows_out_shape, table.dtype),
        scratch_types=[
            pltpu.VMEM((b_per_w,), jnp.int32),
            pltpu.VMEM((b_per_w, d), table.dtype),
            pltpu.SemaphoreType.DMA,
        ],
    )
    def k(table_hbm, idx_hbm, out_hbm, idx_v, rows_v, sem):
        wid = lax.axis_index("s") * info.num_cores + lax.axis_index("c")
        base = wid * b_per_w
        pltpu.sync_copy(idx_hbm.at[pl.ds(base, b_per_w)], idx_v)
        pltpu.async_copy(table_hbm.at[idx_v], rows_v, sem).wait()
        pltpu.sync_copy(rows_v, out_hbm.at[pl.ds(base, b_per_w)])

    return k(table, idx)


def _top9_body(g_ref, bsel_ref, out_ref):
    v = g_ref[...]
    lane = lax.broadcasted_iota(jnp.int32, (1, 128), 1)
    chunks = []
    for k in range(KSEL):
        b = bsel_ref[:, k:k + 1] // B
        chunks.append(b * 128 + lane)
    gidx = jnp.concatenate(chunks, axis=1)
    outs = []
    for r in range(NNBR):
        m = jnp.max(v, axis=1, keepdims=True)
        g = jnp.min(jnp.where(v == m, gidx, N_PAD), axis=1, keepdims=True)
        outs.append(g)
        v = jnp.where(gidx == g, NEG_INF, v)
    outs += [outs[0]] * (KSEL - NNBR)
    out_ref[...] = jnp.concatenate(outs, axis=1)


def _top9_call(gath2d, bsel):
    return pl.pallas_call(
        _top9_body,
        in_specs=[
            pl.BlockSpec((B, KSEL * 128), lambda: (0, 0)),
            pl.BlockSpec((B, KSEL), lambda: (0, 0)),
        ],
        out_specs=pl.BlockSpec((B, KSEL), lambda: (0, 0)),
        out_shape=jax.ShapeDtypeStruct((B, KSEL), jnp.int32),
    )(gath2d, bsel)


RNUM = SAMPLE_NUM - B
NCAND = KSEL * B
CPAD = 9216


def _prefix_body(cand_ref, pos_ref, lab2_ref, lab_ref, p_ref, q_ref,
                 inv_ref, fo_ref):
    cand = cand_ref[...]
    pos = pos_ref[...]
    candnp = cand * (1.0 - pos)
    jr = lax.broadcasted_iota(jnp.int32, (NBLK, 128), 0)
    jc = lax.broadcasted_iota(jnp.int32, (NBLK, 128), 1)
    valid = (jr * 128 + jc) < N
    fb = jnp.where(valid, (1.0 - cand) * (1.0 - pos), 0.0)

    ia = lax.broadcasted_iota(jnp.int32, (NBLK, NBLK), 0)
    ib = lax.broadcasted_iota(jnp.int32, (NBLK, NBLK), 1)
    slt = (ia > ib).astype(jnp.float32)
    ua = lax.broadcasted_iota(jnp.int32, (128, 128), 0)
    ub = lax.broadcasted_iota(jnp.int32, (128, 128), 1)
    su = (ua < ub).astype(jnp.float32)
    ones = jnp.ones((128, 128), jnp.float32)

    def mm(a, b):
        return jax.lax.dot_general(a, b, (((1,), (0,)), ((), ())),
                                   preferred_element_type=jnp.float32)

    p_ref[...] = mm(slt, mm(candnp, ones)) + mm(candnp, su)
    q_ref[...] = mm(slt, mm(fb, ones)) + mm(fb, su)

    la = lab2_ref[...]
    lb = lab_ref[...].reshape(1, B)
    pa = lax.broadcasted_iota(jnp.int32, (B, B), 0)
    pb = lax.broadcasted_iota(jnp.int32, (B, B), 1)
    earlier = ((la == lb) & (pa < pb)).astype(jnp.float32)
    fo = jnp.sum(earlier, axis=0) == 0.0
    lt = (la < lb).astype(jnp.float32)
    fo8 = jnp.broadcast_to(fo.astype(jnp.float32).reshape(1, B), (8, B))
    invm = mm(fo8, lt)
    inv_ref[...] = invm[0].astype(jnp.int32)
    fo_ref[...] = fo.astype(jnp.int32)


def _prefix_call(cand2d, pos2d, labels2d, labels):
    return pl.pallas_call(
        _prefix_body,
        in_specs=[
            pl.BlockSpec((NBLK, 128), lambda: (0, 0)),
            pl.BlockSpec((NBLK, 128), lambda: (0, 0)),
            pl.BlockSpec((B, 1), lambda: (0, 0)),
            pl.BlockSpec((B,), lambda: (0,)),
        ],
        out_specs=[
            pl.BlockSpec((NBLK, 128), lambda: (0, 0)),
            pl.BlockSpec((NBLK, 128), lambda: (0, 0)),
            pl.BlockSpec((B,), lambda: (0,)),
            pl.BlockSpec((B,), lambda: (0,)),
        ],
        out_shape=[
            jax.ShapeDtypeStruct((NBLK, 128), jnp.float32),
            jax.ShapeDtypeStruct((NBLK, 128), jnp.float32),
            jax.ShapeDtypeStruct((B,), jnp.int32),
            jax.ShapeDtypeStruct((B,), jnp.int32),
        ],
    )(cand2d, pos2d, labels2d, labels)


GI = 512


def _rank_body(a2_ref, i2_ref, af_ref, if_ref, r_ref):
    ai = a2_ref[...]
    ii = i2_ref[...]
    aj = af_ref[...].reshape(1, CPAD)
    ij = if_ref[...].reshape(1, CPAD)
    gt = (aj > ai) | ((aj == ai) & (ij < ii))
    r_ref[...] = jnp.sum(gt.astype(jnp.float32), axis=1,
                         keepdims=True).astype(jnp.int32)


def _rank_call(aggc, candidx):
    return pl.pallas_call(
        _rank_body,
        grid=(CPAD // GI,),
        in_specs=[
            pl.BlockSpec((GI, 1), lambda t: (t, 0)),
            pl.BlockSpec((GI, 1), lambda t: (t, 0)),
            pl.BlockSpec((CPAD,), lambda t: (0,)),
            pl.BlockSpec((CPAD,), lambda t: (0,)),
        ],
        out_specs=pl.BlockSpec((GI, 1), lambda t: (t, 0)),
        out_shape=jax.ShapeDtypeStruct((CPAD, 1), jnp.int32),
    )(aggc.reshape(CPAD, 1), candidx.reshape(CPAD, 1), aggc, candidx)


def _sc_masks(topidx_flat, labels):
    mesh = plsc.VectorSubcoreMesh(core_axis_name="c", subcore_axis_name="s")
    info = plsc.get_sparse_core_info()

    @functools.partial(
        pl.kernel, mesh=mesh,
        compiler_params=pltpu.CompilerParams(needs_layout_passes=False),
        out_type=[
            jax.ShapeDtypeStruct((N_PAD,), jnp.float32),
            jax.ShapeDtypeStruct((N_PAD,), jnp.float32),
        ],
        scratch_types=[
            pltpu.VMEM((N_PAD,), jnp.float32),
            pltpu.VMEM((NCAND,), jnp.int32),
        ],
    )
    def k(ti_hbm, lab_hbm, cand_hbm, pos_hbm, mask_v, idx_v):
        wid = lax.axis_index("s") * info.num_cores + lax.axis_index("c")
        ones16 = jnp.ones((16,), jnp.float32)

        @pl.when(wid == 0)
        def _():
            def zf(i, _):
                mask_v[pl.ds(i * 16, 16)] = jnp.zeros((16,), jnp.float32)
                return 0
            lax.fori_loop(0, N_PAD // 16, zf, 0)
            pltpu.sync_copy(ti_hbm, idx_v)

            def sc(i, _):
                iv = idx_v[pl.ds(i * 16, 16)]
                plsc.store_scatter(mask_v, [iv], ones16)
                return 0
            lax.fori_loop(0, NCAND // 16, sc, 0)
            pltpu.sync_copy(mask_v, cand_hbm)

        @pl.when(wid == 1)
        def _():
            def zf(i, _):
                mask_v[pl.ds(i * 16, 16)] = jnp.zeros((16,), jnp.float32)
                return 0
            lax.fori_loop(0, N_PAD // 16, zf, 0)
            pltpu.sync_copy(lab_hbm, idx_v.at[pl.ds(0, B)])

            def sc(i, _):
                iv = idx_v[pl.ds(i * 16, 16)]
                plsc.store_scatter(mask_v, [iv], ones16)
                return 0
            lax.fori_loop(0, B // 16, sc, 0)
            pltpu.sync_copy(mask_v, pos_hbm)

    return k(topidx_flat, labels)


FCH = 14336
NCHUNK = N_PAD // FCH


def _sc_compact(cand, pos, pflat, qflat, agg, labels, fo, inv, cvec, fvec):
    mesh = plsc.VectorSubcoreMesh(core_axis_name="c", subcore_axis_name="s")
    info = plsc.get_sparse_core_info()

    @functools.partial(
        pl.kernel, mesh=mesh,
        compiler_params=pltpu.CompilerParams(needs_layout_passes=False),
        out_type=[
            jax.ShapeDtypeStruct((CPAD,), jnp.float32),
            jax.ShapeDtypeStruct((CPAD,), jnp.int32),
            jax.ShapeDtypeStruct((SAMPLE_NUM,), jnp.int32),
        ],
        scratch_types=[
            pltpu.VMEM((CPAD,), jnp.float32),
            pltpu.VMEM((CPAD,), jnp.int32),
            pltpu.VMEM((SAMPLE_NUM,), jnp.int32),
            pltpu.VMEM((FCH,), jnp.float32),
            pltpu.VMEM((FCH,), jnp.float32),
            pltpu.VMEM((FCH,), jnp.int32),
            pltpu.VMEM((FCH,), jnp.float32),
            pltpu.VMEM((B,), jnp.int32),
            pltpu.VMEM((B,), jnp.int32),
            pltpu.VMEM((B,), jnp.int32),
            pltpu.VMEM((16,), jnp.int32),
            pltpu.VMEM((16,), jnp.int32),
            pltpu.SemaphoreType.DMA,
        ],
    )
    def k(cand_h, pos_h, p_h, q_h, agg_h, lab_h, fo_h, inv_h, cv_h, fv_h,
          aggc_o, candidx_o, rows_o,
          aggc_v, cidx_v, rows_v, cc_v, pc_v, pq_v, ac_v,
          lab_v, fo_v, inv_v, cvec_v, fvec_v, sem):
        wid = lax.axis_index("s") * info.num_cores + lax.axis_index("c")

        @pl.when(wid == 0)
        def _():
            def init(i, _):
                aggc_v[pl.ds(i * 16, 16)] = jnp.full((16,), NEG_INF, jnp.float32)
                cidx_v[pl.ds(i * 16, 16)] = jnp.full((16,), -1, jnp.int32)
                return 0
            lax.fori_loop(0, CPAD // 16, init, 0)

            for c in range(NCHUNK):
                sl = pl.ds(c * FCH, FCH)
                h1 = pltpu.async_copy(cand_h.at[sl], cc_v, sem)
                h2 = pltpu.async_copy(pos_h.at[sl], pc_v, sem)
                h3 = pltpu.async_copy(p_h.at[sl], pq_v, sem)
                h4 = pltpu.async_copy(agg_h.at[sl], ac_v, sem)
                h1.wait(); h2.wait(); h3.wait(); h4.wait()
                base = c * FCH

                def sweep(i, _):
                    jv = lax.iota(jnp.int32, 16) + (base + i * 16)
                    cv = cc_v[pl.ds(i * 16, 16)]
                    pv = pc_v[pl.ds(i * 16, 16)]
                    Pv = pq_v[pl.ds(i * 16, 16)]
                    av = ac_v[pl.ds(i * 16, 16)]
                    isc = (cv > 0.0) & (pv == 0.0)
                    Pc = jnp.minimum(Pv, CPAD - 1)
                    plsc.store_scatter(aggc_v, [Pc], av, mask=isc)
                    plsc.store_scatter(cidx_v, [Pc], jv, mask=isc)
                    return 0
                lax.fori_loop(0, FCH // 16, sweep, 0)

            pltpu.sync_copy(aggc_v, aggc_o)
            pltpu.sync_copy(cidx_v, candidx_o)

        @pl.when(wid == 1)
        def _():
            def zrow(i, _):
                rows_v[pl.ds(i * 16, 16)] = jnp.zeros((16,), jnp.int32)
                return 0
            lax.fori_loop(0, B // 16, zrow, 0)

            pltpu.sync_copy(lab_h, lab_v)
            pltpu.sync_copy(fo_h, fo_v)
            pltpu.sync_copy(inv_h, inv_v)
            pltpu.sync_copy(cv_h, cvec_v)
            pltpu.sync_copy(fv_h, fvec_v)

            def usc(i, _):
                lab = lab_v[pl.ds(i * 16, 16)]
                dst = inv_v[pl.ds(i * 16, 16)]
                m = fo_v[pl.ds(i * 16, 16)] != 0
                plsc.store_scatter(rows_v, [jnp.minimum(dst, B - 1)], lab,
                                   mask=m)
                return 0
            lax.fori_loop(0, B // 16, usc, 0)

            cvec = cvec_v[pl.ds(0, 16)]
            fvec = fvec_v[pl.ds(0, 16)]
            for c in range(NCHUNK):
                sl = pl.ds(c * FCH, FCH)
                h1 = pltpu.async_copy(cand_h.at[sl], cc_v, sem)
                h2 = pltpu.async_copy(pos_h.at[sl], pc_v, sem)
                h3 = pltpu.async_copy(q_h.at[sl], pq_v, sem)
                h1.wait(); h2.wait(); h3.wait()
                base = c * FCH

                def fsw(i, _):
                    jv = lax.iota(jnp.int32, 16) + (base + i * 16)
                    cv = cc_v[pl.ds(i * 16, 16)]
                    pv = pc_v[pl.ds(i * 16, 16)]
                    Qv = pq_v[pl.ds(i * 16, 16)]
                    fbm = ((cv == 0.0) & (pv == 0.0) & (jv < N)
                           & (Qv < fvec))
                    dst = jnp.minimum(cvec + Qv, SAMPLE_NUM - 1)
                    plsc.store_scatter(rows_v, [dst], jv, mask=fbm)
                    return 0
                lax.fori_loop(0, FCH // 16, fsw, 0)

            pltpu.sync_copy(rows_v, rows_o)

    return k(cand, pos, pflat, qflat, agg, labels, fo, inv, cvec, fvec)


def _sc_rows(rows_part, rank, candidx):
    mesh = plsc.VectorSubcoreMesh(core_axis_name="c", subcore_axis_name="s")
    info = plsc.get_sparse_core_info()

    @functools.partial(
        pl.kernel, mesh=mesh,
        compiler_params=pltpu.CompilerParams(needs_layout_passes=False),
        out_type=jax.ShapeDtypeStruct((SAMPLE_NUM,), jnp.int32),
        scratch_types=[
            pltpu.VMEM((SAMPLE_NUM,), jnp.int32),
            pltpu.VMEM((CPAD,), jnp.int32),
            pltpu.VMEM((CPAD,), jnp.int32),
        ],
    )
    def k(rp_h, rk_h, ci_h, rows_o, rows_v, rk_v, ci_v):
        wid = lax.axis_index("s") * info.num_cores + lax.axis_index("c")

        @pl.when(wid == 0)
        def _():
            pltpu.sync_copy(rp_h, rows_v)
            pltpu.sync_copy(rk_h, rk_v)
            pltpu.sync_copy(ci_h, ci_v)

            def sc(i, _):
                rv = rk_v[pl.ds(i * 16, 16)]
                cv = ci_v[pl.ds(i * 16, 16)]
                m = (rv < RNUM) & (cv >= 0)
                dst = B + jnp.minimum(rv, RNUM - 1)
                plsc.store_scatter(rows_v, [dst], cv, mask=m)
                return 0
            lax.fori_loop(0, CPAD // 16, sc, 0)
            pltpu.sync_copy(rows_v, rows_o)

    return k(rows_part, rank, candidx)


def kernel(features, labels, weight):
    bsz = features.shape[0]

    fn = features / (jnp.linalg.norm(features, axis=1, keepdims=True) + 1e-12)
    denom = jnp.linalg.norm(weight, axis=1) + 1e-12
    denom = jnp.pad(denom, (0, N_PAD - N), constant_values=1.0)

    scores, m1t, agg = _scores_call(fn, weight, denom)

    selT = _blocksel_call(m1t)
    sel = selT.T
    gath = _sc_gather(scores.reshape(NBLK * B, 128), sel.reshape(-1),
                      (B * KSEL, 128))
    topidx = _top9_call(gath.reshape(B, KSEL * 128), sel)

    cand, pos = _sc_masks(topidx.reshape(-1), labels)

    p2d, q2d, inv, fo = _prefix_call(cand.reshape(NBLK, 128),
                                     pos.reshape(NBLK, 128),
                                     labels.reshape(B, 1), labels)
    pflat = p2d.reshape(-1).astype(jnp.int32)
    qflat = q2d.reshape(-1).astype(jnp.int32)
    c_tot = pflat[-1]
    f_tot = jnp.maximum(RNUM - c_tot, 0)
    cvec = jnp.full((16,), B, jnp.int32) + c_tot
    fvec = jnp.full((16,), 0, jnp.int32) + f_tot

    aggc, candidx, rows_part = _sc_compact(
        cand, pos, pflat, qflat, agg, labels, fo, inv, cvec, fvec)

    rank = _rank_call(aggc, candidx).reshape(-1)

    rows = _sc_rows(rows_part, rank, candidx)

    w_sel = _sc_gather(weight, rows, (SAMPLE_NUM, FDIM))
    bias = jnp.zeros((SAMPLE_NUM,), jnp.float32)
    return w_sel, bias, inv.astype(jnp.int64)
